# Initial kernel scaffold; baseline (speedup 1.0000x reference)
import jax
import jax.numpy as jnp
from jax.experimental import pallas as pl


def kernel(x, Wg, Wl1, Wl2):
    raise NotImplementedError("write your pallas kernel here")


# speedup vs baseline: 4.6128x; 4.6128x over previous
import functools

import jax
import jax.numpy as jnp
from jax import lax
from jax.experimental import pallas as pl
from jax.experimental.pallas import tpu as pltpu
from jax.experimental.pallas import tpu_sc as plsc

D = 768
H = 1024
E = 64
M = 2048
C = 128
HC = 512
NHC = H // HC
NW = 32
TPW = M // NW
CH = 32
DUMP = E * C


def _router_body(x_ref, wg_ref, ridx_ref, wrt_ref):
    x = x_ref[...]
    logits = lax.dot_general(x, wg_ref[...], (((1,), (0,)), ((), ())),
                             preferred_element_type=jnp.float32)
    it = lax.broadcasted_iota(jnp.int32, (M, E), 1)
    m1 = jnp.max(logits, axis=1, keepdims=True)
    i1 = jnp.min(jnp.where(logits == m1, it, E), axis=1, keepdims=True)
    l2 = jnp.where(it == i1, -jnp.inf, logits)
    m2 = jnp.max(l2, axis=1, keepdims=True)
    i2 = jnp.min(jnp.where(l2 == m2, it, E), axis=1, keepdims=True)
    w2 = jax.nn.sigmoid(m2 - m1)
    w1 = 1.0 - w2

    one = jnp.float32(1.0)
    zero = jnp.float32(0.0)
    cnt = jnp.where(it == i1, one, zero) + jnp.where(it == i2, one, zero)

    B = 256
    rio = lax.broadcasted_iota(jnp.int32, (B, B), 0)
    cio = lax.broadcasted_iota(jnp.int32, (B, B), 1)
    tri = jnp.where(rio > cio, one, zero)
    carry = jnp.zeros((1, E), jnp.float32)
    blocks = []
    for b in range(M // B):
        blk = lax.slice(cnt, (b * B, 0), ((b + 1) * B, E))
        blocks.append(
            lax.dot_general(tri, blk, (((1,), (0,)), ((), ())),
                            preferred_element_type=jnp.float32) + carry)
        carry = carry + jnp.sum(blk, axis=0, keepdims=True)
    csum = jnp.concatenate(blocks, axis=0)

    s1 = jnp.sum(jnp.where(it == i1, csum, zero), axis=1).astype(jnp.int32)
    s2 = jnp.sum(jnp.where(it == i2, csum, zero), axis=1).astype(jnp.int32)
    e1 = i1[:, 0]
    e2 = i2[:, 0]
    r1 = jnp.where(s1 < C, e1 * C + s1, DUMP)
    r2 = jnp.where(s2 < C, e2 * C + s2, DUMP)
    ridx_ref[...] = jnp.concatenate(
        [r1.reshape(1, M), r2.reshape(1, M)], axis=0)

    itE = lax.broadcasted_iota(jnp.int32, (M, E + 1), 1)
    itC = lax.broadcasted_iota(jnp.int32, (M, C), 1)
    lhs1 = jnp.where(itE == i1, w1, zero)
    lhs2 = jnp.where(itE == i2, w2, zero)
    rhs1 = jnp.where(itC == s1[:, None], one, zero)
    rhs2 = jnp.where(itC == s2[:, None], one, zero)
    wrt = (lax.dot_general(rhs1, lhs1, (((0,), (0,)), ((), ())),
                           preferred_element_type=jnp.float32) +
           lax.dot_general(rhs2, lhs2, (((0,), (0,)), ((), ())),
                           preferred_element_type=jnp.float32))
    wrt_ref[...] = wrt


def _router(x, Wg):
    return pl.pallas_call(
        _router_body,
        out_shape=(jax.ShapeDtypeStruct((2, M), jnp.int32),
                   jax.ShapeDtypeStruct((C, E + 1), jnp.float32)),
    )(x, Wg)


def _dispatch_body(x_hbm, ridx_hbm, xg_hbm, idx0, idx1, xbuf, sem0, sem1):
    wid = lax.axis_index("s") * 2 + lax.axis_index("c")
    for c in range(TPW // CH):
        base = wid * TPW + c * CH
        pltpu.sync_copy(ridx_hbm.at[pl.ds(base, CH)], idx0)
        pltpu.sync_copy(ridx_hbm.at[pl.ds(M + base, CH)], idx1)
        pltpu.sync_copy(x_hbm.at[pl.ds(base, CH)], xbuf)
        cp0 = pltpu.make_async_copy(xbuf, xg_hbm.at[idx0], sem0)
        cp1 = pltpu.make_async_copy(xbuf, xg_hbm.at[idx1], sem1)
        cp0.start()
        cp1.start()
        cp0.wait()
        cp1.wait()


@functools.cache
def _dispatch():
    return pl.kernel(
        _dispatch_body,
        out_type=jax.ShapeDtypeStruct((E * C + C, D), jnp.float32),
        mesh=plsc.VectorSubcoreMesh(core_axis_name="c", subcore_axis_name="s"),
        scratch_types=[
            pltpu.VMEM((CH,), jnp.int32),
            pltpu.VMEM((CH,), jnp.int32),
            pltpu.VMEM((CH, D), jnp.float32),
            pltpu.SemaphoreType.DMA,
            pltpu.SemaphoreType.DMA,
        ],
    )


def _ffn_body(xg_ref, wrt_ref, w1a_ref, w1b_ref, w2_ref, yw_ref):
    e = pl.program_id(0)
    hc = pl.program_id(1)
    lane = lax.broadcasted_iota(jnp.int32, (C, E + 1), 1)
    col = jnp.sum(jnp.where(lane == e, wrt_ref[...], 0.0), axis=1,
                  keepdims=True)
    xm = jnp.where(col > 0.0, xg_ref[...], 0.0)
    a = lax.dot_general(xm, w1a_ref[0], (((1,), (0,)), ((), ())),
                        preferred_element_type=jnp.float32)
    b = lax.dot_general(xm, w1b_ref[0], (((1,), (0,)), ((), ())),
                        preferred_element_type=jnp.float32)
    h = b * jax.nn.sigmoid(b) * a
    y = lax.dot_general(h, w2_ref[0], (((1,), (0,)), ((), ())),
                        preferred_element_type=jnp.float32)

    @pl.when(hc == 0)
    def _():
        yw_ref[...] = y

    @pl.when(hc > 0)
    def _():
        yw_ref[...] = yw_ref[...] + y

    @pl.when(hc == NHC - 1)
    def _():
        yw_ref[...] = yw_ref[...] * col


def _ffn(xg, wrt, Wl1, Wl2):
    wcap = lambda e, hc: jnp.minimum(e, E - 1)
    return pl.pallas_call(
        _ffn_body,
        grid=(E + 1, NHC),
        in_specs=[
            pl.BlockSpec((C, D), lambda e, hc: (e, 0)),
            pl.BlockSpec((C, E + 1), lambda e, hc: (0, 0)),
            pl.BlockSpec((1, D, HC), lambda e, hc: (wcap(e, hc), 0, hc)),
            pl.BlockSpec((1, D, HC), lambda e, hc: (wcap(e, hc), 0, NHC + hc)),
            pl.BlockSpec((1, HC, D), lambda e, hc: (wcap(e, hc), hc, 0)),
        ],
        out_specs=pl.BlockSpec((C, D), lambda e, hc: (e, 0)),
        out_shape=jax.ShapeDtypeStruct((E * C + C, D), jnp.float32),
    )(xg, wrt, Wl1, Wl1, Wl2)


def _combine_body(yw_hbm, ridx_hbm, out_hbm, idx0, idx1, buf0, buf1,
                  sem0, sem1):
    wid = lax.axis_index("s") * 2 + lax.axis_index("c")
    for c in range(TPW // CH):
        base = wid * TPW + c * CH
        pltpu.sync_copy(ridx_hbm.at[pl.ds(base, CH)], idx0)
        pltpu.sync_copy(ridx_hbm.at[pl.ds(M + base, CH)], idx1)
        cp0 = pltpu.make_async_copy(yw_hbm.at[idx0], buf0, sem0)
        cp1 = pltpu.make_async_copy(yw_hbm.at[idx1], buf1, sem1)
        cp0.start()
        cp1.start()
        cp0.wait()
        cp1.wait()

        def add_row(i, _):
            for j in range(D // 16):
                sl = pl.ds(j * 16, 16)
                buf0[i, sl] = buf0[i, sl] + buf1[i, sl]
            return 0

        lax.fori_loop(0, CH, add_row, 0)
        pltpu.sync_copy(buf0, out_hbm.at[pl.ds(base, CH)])


@functools.cache
def _combine():
    return pl.kernel(
        _combine_body,
        out_type=jax.ShapeDtypeStruct((M, D), jnp.float32),
        mesh=plsc.VectorSubcoreMesh(core_axis_name="c", subcore_axis_name="s"),
        scratch_types=[
            pltpu.VMEM((CH,), jnp.int32),
            pltpu.VMEM((CH,), jnp.int32),
            pltpu.VMEM((CH, D), jnp.float32),
            pltpu.VMEM((CH, D), jnp.float32),
            pltpu.SemaphoreType.DMA,
            pltpu.SemaphoreType.DMA,
        ],
    )


def kernel(x, Wg, Wl1, Wl2):
    ridx, wrt = _router(x, Wg)
    ridx_flat = ridx.reshape(2 * M)
    xg = _dispatch()(x, ridx_flat)
    yw = _ffn(xg, wrt, Wl1, Wl2)
    return _combine()(yw, ridx_flat)


# speedup vs baseline: 5.2059x; 1.1286x over previous
import functools

import jax
import jax.numpy as jnp
from jax import lax
from jax.experimental import pallas as pl
from jax.experimental.pallas import tpu as pltpu
from jax.experimental.pallas import tpu_sc as plsc

D = 768
H = 1024
E = 64
M = 2048
C = 128
HC = 1024
NHC = H // HC
NW = 32
TPW = M // NW
CH = 32
DUMP = E * C


def _router_body(x_ref, wg_ref, ridx_ref, wrt_ref):
    x = x_ref[...]
    logits = lax.dot_general(x, wg_ref[...], (((1,), (0,)), ((), ())),
                             preferred_element_type=jnp.float32)
    it = lax.broadcasted_iota(jnp.int32, (M, E), 1)
    m1 = jnp.max(logits, axis=1, keepdims=True)
    i1 = jnp.min(jnp.where(logits == m1, it, E), axis=1, keepdims=True)
    l2 = jnp.where(it == i1, -jnp.inf, logits)
    m2 = jnp.max(l2, axis=1, keepdims=True)
    i2 = jnp.min(jnp.where(l2 == m2, it, E), axis=1, keepdims=True)
    w2 = jax.nn.sigmoid(m2 - m1)
    w1 = 1.0 - w2

    one = jnp.float32(1.0)
    zero = jnp.float32(0.0)
    cnt = jnp.where(it == i1, one, zero) + jnp.where(it == i2, one, zero)

    B = 256
    rio = lax.broadcasted_iota(jnp.int32, (B, B), 0)
    cio = lax.broadcasted_iota(jnp.int32, (B, B), 1)
    tri = jnp.where(rio > cio, one, zero)
    carry = jnp.zeros((1, E), jnp.float32)
    blocks = []
    for b in range(M // B):
        blk = lax.slice(cnt, (b * B, 0), ((b + 1) * B, E))
        blocks.append(
            lax.dot_general(tri, blk, (((1,), (0,)), ((), ())),
                            preferred_element_type=jnp.float32) + carry)
        carry = carry + jnp.sum(blk, axis=0, keepdims=True)
    csum = jnp.concatenate(blocks, axis=0)

    s1 = jnp.sum(jnp.where(it == i1, csum, zero), axis=1).astype(jnp.int32)
    s2 = jnp.sum(jnp.where(it == i2, csum, zero), axis=1).astype(jnp.int32)
    e1 = i1[:, 0]
    e2 = i2[:, 0]
    r1 = jnp.where(s1 < C, e1 * C + s1, DUMP)
    r2 = jnp.where(s2 < C, e2 * C + s2, DUMP)
    ridx_ref[...] = jnp.concatenate(
        [r1.reshape(1, M), r2.reshape(1, M)], axis=0)

    itE = lax.broadcasted_iota(jnp.int32, (M, E + 1), 1)
    itC = lax.broadcasted_iota(jnp.int32, (M, C), 1)
    lhs1 = jnp.where(itE == i1, w1, zero)
    lhs2 = jnp.where(itE == i2, w2, zero)
    rhs1 = jnp.where(itC == s1[:, None], one, zero)
    rhs2 = jnp.where(itC == s2[:, None], one, zero)
    wrt = (lax.dot_general(rhs1, lhs1, (((0,), (0,)), ((), ())),
                           preferred_element_type=jnp.float32) +
           lax.dot_general(rhs2, lhs2, (((0,), (0,)), ((), ())),
                           preferred_element_type=jnp.float32))
    wrt_ref[...] = wrt


def _router(x, Wg):
    return pl.pallas_call(
        _router_body,
        out_shape=(jax.ShapeDtypeStruct((2, M), jnp.int32),
                   jax.ShapeDtypeStruct((C, E + 1), jnp.float32)),
    )(x, Wg)


def _dispatch_body(x_hbm, ridx_hbm, xg_hbm, idx0, idx1, xbuf, sem0, sem1):
    wid = lax.axis_index("s") * 2 + lax.axis_index("c")
    for c in range(TPW // CH):
        base = wid * TPW + c * CH
        pltpu.sync_copy(ridx_hbm.at[pl.ds(base, CH)], idx0)
        pltpu.sync_copy(ridx_hbm.at[pl.ds(M + base, CH)], idx1)
        pltpu.sync_copy(x_hbm.at[pl.ds(base, CH)], xbuf)
        cp0 = pltpu.make_async_copy(xbuf, xg_hbm.at[idx0], sem0)
        cp1 = pltpu.make_async_copy(xbuf, xg_hbm.at[idx1], sem1)
        cp0.start()
        cp1.start()
        cp0.wait()
        cp1.wait()


@functools.cache
def _dispatch():
    return pl.kernel(
        _dispatch_body,
        out_type=jax.ShapeDtypeStruct((E * C + C, D), jnp.float32),
        mesh=plsc.VectorSubcoreMesh(core_axis_name="c", subcore_axis_name="s"),
        scratch_types=[
            pltpu.VMEM((CH,), jnp.int32),
            pltpu.VMEM((CH,), jnp.int32),
            pltpu.VMEM((CH, D), jnp.float32),
            pltpu.SemaphoreType.DMA,
            pltpu.SemaphoreType.DMA,
        ],
    )


def _ffn_body(xg_ref, wrt_ref, w1a_ref, w1b_ref, w2_ref, yw_ref):
    e = pl.program_id(0)
    lane = lax.broadcasted_iota(jnp.int32, (C, E + 1), 1)
    col = jnp.sum(jnp.where(lane == e, wrt_ref[...], 0.0), axis=1,
                  keepdims=True)
    xm = jnp.where(col > 0.0, xg_ref[...], 0.0)
    a = lax.dot_general(xm, w1a_ref[0], (((1,), (0,)), ((), ())),
                        preferred_element_type=jnp.float32)
    b = lax.dot_general(xm, w1b_ref[0], (((1,), (0,)), ((), ())),
                        preferred_element_type=jnp.float32)
    h = b * jax.nn.sigmoid(b) * a
    y = lax.dot_general(h, w2_ref[0], (((1,), (0,)), ((), ())),
                        preferred_element_type=jnp.float32)
    yw_ref[...] = y * col


def _ffn(xg, wrt, Wl1, Wl2):
    wcap = lambda e: jnp.minimum(e, E - 1)
    return pl.pallas_call(
        _ffn_body,
        grid=(E + 1,),
        in_specs=[
            pl.BlockSpec((C, D), lambda e: (e, 0)),
            pl.BlockSpec((C, E + 1), lambda e: (0, 0)),
            pl.BlockSpec((1, D, HC), lambda e: (wcap(e), 0, 0)),
            pl.BlockSpec((1, D, HC), lambda e: (wcap(e), 0, 1)),
            pl.BlockSpec((1, HC, D), lambda e: (wcap(e), 0, 0)),
        ],
        out_specs=pl.BlockSpec((C, D), lambda e: (e, 0)),
        out_shape=jax.ShapeDtypeStruct((E * C + C, D), jnp.float32),
    )(xg, wrt, Wl1, Wl1, Wl2)


def _combine_body(yw_hbm, ridx_hbm, out_hbm, idx0, idx1, buf0, buf1,
                  sem0, sem1):
    wid = lax.axis_index("s") * 2 + lax.axis_index("c")
    for c in range(TPW // CH):
        base = wid * TPW + c * CH
        pltpu.sync_copy(ridx_hbm.at[pl.ds(base, CH)], idx0)
        pltpu.sync_copy(ridx_hbm.at[pl.ds(M + base, CH)], idx1)
        cp0 = pltpu.make_async_copy(yw_hbm.at[idx0], buf0, sem0)
        cp1 = pltpu.make_async_copy(yw_hbm.at[idx1], buf1, sem1)
        cp0.start()
        cp1.start()
        cp0.wait()
        cp1.wait()

        def add_row(i, _):
            for j in range(D // 16):
                sl = pl.ds(j * 16, 16)
                buf0[i, sl] = buf0[i, sl] + buf1[i, sl]
            return 0

        lax.fori_loop(0, CH, add_row, 0)
        pltpu.sync_copy(buf0, out_hbm.at[pl.ds(base, CH)])


@functools.cache
def _combine():
    return pl.kernel(
        _combine_body,
        out_type=jax.ShapeDtypeStruct((M, D), jnp.float32),
        mesh=plsc.VectorSubcoreMesh(core_axis_name="c", subcore_axis_name="s"),
        scratch_types=[
            pltpu.VMEM((CH,), jnp.int32),
            pltpu.VMEM((CH,), jnp.int32),
            pltpu.VMEM((CH, D), jnp.float32),
            pltpu.VMEM((CH, D), jnp.float32),
            pltpu.SemaphoreType.DMA,
            pltpu.SemaphoreType.DMA,
        ],
    )


def kernel(x, Wg, Wl1, Wl2):
    ridx, wrt = _router(x, Wg)
    ridx_flat = ridx.reshape(2 * M)
    xg = _dispatch()(x, ridx_flat)
    yw = _ffn(xg, wrt, Wl1, Wl2)
    return _combine()(yw, ridx_flat)


# speedup vs baseline: 5.2103x; 1.0008x over previous
import functools

import jax
import jax.numpy as jnp
from jax import lax
from jax.experimental import pallas as pl
from jax.experimental.pallas import tpu as pltpu
from jax.experimental.pallas import tpu_sc as plsc

D = 768
H = 1024
E = 64
M = 2048
C = 128
HC = 1024
NHC = H // HC
NW = 32
TPW = M // NW
CH = 32
DUMP = E * C


def _router_body(x_ref, wg_ref, ridx_ref, wrt_ref):
    x = x_ref[...]
    logits = lax.dot_general(x, wg_ref[...], (((1,), (0,)), ((), ())),
                             preferred_element_type=jnp.float32)
    it = lax.broadcasted_iota(jnp.int32, (M, E), 1)
    m1 = jnp.max(logits, axis=1, keepdims=True)
    i1 = jnp.min(jnp.where(logits == m1, it, E), axis=1, keepdims=True)
    l2 = jnp.where(it == i1, -jnp.inf, logits)
    m2 = jnp.max(l2, axis=1, keepdims=True)
    i2 = jnp.min(jnp.where(l2 == m2, it, E), axis=1, keepdims=True)
    w2 = jax.nn.sigmoid(m2 - m1)
    w1 = 1.0 - w2

    one = jnp.float32(1.0)
    zero = jnp.float32(0.0)
    cnt = jnp.where(it == i1, one, zero) + jnp.where(it == i2, one, zero)

    B = 256
    rio = lax.broadcasted_iota(jnp.int32, (B, B), 0)
    cio = lax.broadcasted_iota(jnp.int32, (B, B), 1)
    tri = jnp.where(rio > cio, one, zero)
    carry = jnp.zeros((1, E), jnp.float32)
    blocks = []
    for b in range(M // B):
        blk = lax.slice(cnt, (b * B, 0), ((b + 1) * B, E))
        blocks.append(
            lax.dot_general(tri, blk, (((1,), (0,)), ((), ())),
                            preferred_element_type=jnp.float32) + carry)
        carry = carry + jnp.sum(blk, axis=0, keepdims=True)
    csum = jnp.concatenate(blocks, axis=0)

    s1 = jnp.sum(jnp.where(it == i1, csum, zero), axis=1).astype(jnp.int32)
    s2 = jnp.sum(jnp.where(it == i2, csum, zero), axis=1).astype(jnp.int32)
    e1 = i1[:, 0]
    e2 = i2[:, 0]
    r1 = jnp.where(s1 < C, e1 * C + s1, DUMP)
    r2 = jnp.where(s2 < C, e2 * C + s2, DUMP)
    ridx_ref[...] = jnp.concatenate(
        [r1.reshape(1, M), r2.reshape(1, M)], axis=0)

    itE = lax.broadcasted_iota(jnp.int32, (M, E + 1), 1)
    itC = lax.broadcasted_iota(jnp.int32, (M, C), 1)
    lhs1 = jnp.where(itE == i1, w1, zero)
    lhs2 = jnp.where(itE == i2, w2, zero)
    rhs1 = jnp.where(itC == s1[:, None], one, zero)
    rhs2 = jnp.where(itC == s2[:, None], one, zero)
    wrt = (lax.dot_general(rhs1, lhs1, (((0,), (0,)), ((), ())),
                           preferred_element_type=jnp.float32) +
           lax.dot_general(rhs2, lhs2, (((0,), (0,)), ((), ())),
                           preferred_element_type=jnp.float32))
    wrt_ref[...] = wrt


def _router(x, Wg):
    return pl.pallas_call(
        _router_body,
        out_shape=(jax.ShapeDtypeStruct((2, M), jnp.int32),
                   jax.ShapeDtypeStruct((C, E + 1), jnp.float32)),
    )(x, Wg)


def _dispatch_body(x_hbm, ridx_hbm, xg_hbm, idx0, idx1, xbuf, sem0, sem1):
    wid = lax.axis_index("s") * 2 + lax.axis_index("c")
    for c in range(TPW // CH):
        base = wid * TPW + c * CH
        pltpu.sync_copy(ridx_hbm.at[pl.ds(base, CH)], idx0)
        pltpu.sync_copy(ridx_hbm.at[pl.ds(M + base, CH)], idx1)
        pltpu.sync_copy(x_hbm.at[pl.ds(base, CH)], xbuf)
        cp0 = pltpu.make_async_copy(xbuf, xg_hbm.at[idx0], sem0)
        cp1 = pltpu.make_async_copy(xbuf, xg_hbm.at[idx1], sem1)
        cp0.start()
        cp1.start()
        cp0.wait()
        cp1.wait()


@functools.cache
def _dispatch():
    return pl.kernel(
        _dispatch_body,
        out_type=jax.ShapeDtypeStruct((E * C + C, D), jnp.float32),
        mesh=plsc.VectorSubcoreMesh(core_axis_name="c", subcore_axis_name="s"),
        scratch_types=[
            pltpu.VMEM((CH,), jnp.int32),
            pltpu.VMEM((CH,), jnp.int32),
            pltpu.VMEM((CH, D), jnp.float32),
            pltpu.SemaphoreType.DMA,
            pltpu.SemaphoreType.DMA,
        ],
    )


def _ffn_body(xg_ref, wrt_ref, w1_ref, w2_ref, yw_ref):
    e = pl.program_id(0)
    lane = lax.broadcasted_iota(jnp.int32, (C, E + 1), 1)
    col = jnp.sum(jnp.where(lane == e, wrt_ref[...], 0.0), axis=1,
                  keepdims=True)
    xm = jnp.where(col > 0.0, xg_ref[...], 0.0)
    a = lax.dot_general(xm, w1_ref[0, :, :H], (((1,), (0,)), ((), ())),
                        preferred_element_type=jnp.float32)
    b = lax.dot_general(xm, w1_ref[0, :, H:], (((1,), (0,)), ((), ())),
                        preferred_element_type=jnp.float32)
    h = b * jax.nn.sigmoid(b) * a
    y = lax.dot_general(h, w2_ref[0], (((1,), (0,)), ((), ())),
                        preferred_element_type=jnp.float32)
    yw_ref[...] = y * col


def _ffn(xg, wrt, Wl1, Wl2):
    wcap = lambda e: jnp.minimum(e, E - 1)
    return pl.pallas_call(
        _ffn_body,
        grid=(E + 1,),
        in_specs=[
            pl.BlockSpec((C, D), lambda e: (e, 0)),
            pl.BlockSpec((C, E + 1), lambda e: (0, 0)),
            pl.BlockSpec((1, D, 2 * H), lambda e: (wcap(e), 0, 0)),
            pl.BlockSpec((1, H, D), lambda e: (wcap(e), 0, 0)),
        ],
        out_specs=pl.BlockSpec((C, D), lambda e: (e, 0)),
        out_shape=jax.ShapeDtypeStruct((E * C + C, D), jnp.float32),
    )(xg, wrt, Wl1, Wl2)


def _combine_body(yw_hbm, ridx_hbm, out_hbm, idx0, idx1, buf0, buf1,
                  sem0, sem1):
    wid = lax.axis_index("s") * 2 + lax.axis_index("c")
    for c in range(TPW // CH):
        base = wid * TPW + c * CH
        pltpu.sync_copy(ridx_hbm.at[pl.ds(base, CH)], idx0)
        pltpu.sync_copy(ridx_hbm.at[pl.ds(M + base, CH)], idx1)
        cp0 = pltpu.make_async_copy(yw_hbm.at[idx0], buf0, sem0)
        cp1 = pltpu.make_async_copy(yw_hbm.at[idx1], buf1, sem1)
        cp0.start()
        cp1.start()
        cp0.wait()
        cp1.wait()

        def add_row(i, _):
            for j in range(D // 16):
                sl = pl.ds(j * 16, 16)
                buf0[i, sl] = buf0[i, sl] + buf1[i, sl]
            return 0

        lax.fori_loop(0, CH, add_row, 0)
        pltpu.sync_copy(buf0, out_hbm.at[pl.ds(base, CH)])


@functools.cache
def _combine():
    return pl.kernel(
        _combine_body,
        out_type=jax.ShapeDtypeStruct((M, D), jnp.float32),
        mesh=plsc.VectorSubcoreMesh(core_axis_name="c", subcore_axis_name="s"),
        scratch_types=[
            pltpu.VMEM((CH,), jnp.int32),
            pltpu.VMEM((CH,), jnp.int32),
            pltpu.VMEM((CH, D), jnp.float32),
            pltpu.VMEM((CH, D), jnp.float32),
            pltpu.SemaphoreType.DMA,
            pltpu.SemaphoreType.DMA,
        ],
    )


def kernel(x, Wg, Wl1, Wl2):
    ridx, wrt = _router(x, Wg)
    ridx_flat = ridx.reshape(2 * M)
    xg = _dispatch()(x, ridx_flat)
    yw = _ffn(xg, wrt, Wl1, Wl2)
    return _combine()(yw, ridx_flat)


# speedup vs baseline: 5.3185x; 1.0208x over previous
import functools

import jax
import jax.numpy as jnp
from jax import lax
from jax.experimental import pallas as pl
from jax.experimental.pallas import tpu as pltpu
from jax.experimental.pallas import tpu_sc as plsc

D = 768
H = 1024
E = 64
M = 2048
C = 128
HC = 1024
NHC = H // HC
NW = 32
TPW = M // NW
CH = 32
DUMP = E * C


def _router_body(x_ref, wg_ref, ridx_ref, wrt_ref, xp_ref):
    x = x_ref[...]
    xb = x.astype(jnp.bfloat16)
    lo = lax.bitcast_convert_type(xb[:, :D // 2], jnp.uint16).astype(jnp.uint32)
    hi = lax.bitcast_convert_type(xb[:, D // 2:], jnp.uint16).astype(jnp.uint32)
    xp_ref[...] = lax.bitcast_convert_type(lo | (hi << 16), jnp.int32)
    logits = lax.dot_general(x, wg_ref[...], (((1,), (0,)), ((), ())),
                             preferred_element_type=jnp.float32)
    it = lax.broadcasted_iota(jnp.int32, (M, E), 1)
    m1 = jnp.max(logits, axis=1, keepdims=True)
    i1 = jnp.min(jnp.where(logits == m1, it, E), axis=1, keepdims=True)
    l2 = jnp.where(it == i1, -jnp.inf, logits)
    m2 = jnp.max(l2, axis=1, keepdims=True)
    i2 = jnp.min(jnp.where(l2 == m2, it, E), axis=1, keepdims=True)
    w2 = jax.nn.sigmoid(m2 - m1)
    w1 = 1.0 - w2

    one = jnp.float32(1.0)
    zero = jnp.float32(0.0)
    cnt = jnp.where(it == i1, one, zero) + jnp.where(it == i2, one, zero)

    B = 256
    rio = lax.broadcasted_iota(jnp.int32, (B, B), 0)
    cio = lax.broadcasted_iota(jnp.int32, (B, B), 1)
    tri = jnp.where(rio > cio, one, zero)
    carry = jnp.zeros((1, E), jnp.float32)
    blocks = []
    for b in range(M // B):
        blk = lax.slice(cnt, (b * B, 0), ((b + 1) * B, E))
        blocks.append(
            lax.dot_general(tri, blk, (((1,), (0,)), ((), ())),
                            preferred_element_type=jnp.float32) + carry)
        carry = carry + jnp.sum(blk, axis=0, keepdims=True)
    csum = jnp.concatenate(blocks, axis=0)

    s1 = jnp.sum(jnp.where(it == i1, csum, zero), axis=1).astype(jnp.int32)
    s2 = jnp.sum(jnp.where(it == i2, csum, zero), axis=1).astype(jnp.int32)
    e1 = i1[:, 0]
    e2 = i2[:, 0]
    r1 = jnp.where(s1 < C, e1 * C + s1, DUMP)
    r2 = jnp.where(s2 < C, e2 * C + s2, DUMP)
    ridx_ref[...] = jnp.concatenate(
        [r1.reshape(1, M), r2.reshape(1, M)], axis=0)

    itE = lax.broadcasted_iota(jnp.int32, (M, E + 1), 1)
    itC = lax.broadcasted_iota(jnp.int32, (M, C), 1)
    lhs1 = jnp.where(itE == i1, w1, zero)
    lhs2 = jnp.where(itE == i2, w2, zero)
    rhs1 = jnp.where(itC == s1[:, None], one, zero)
    rhs2 = jnp.where(itC == s2[:, None], one, zero)
    wrt = (lax.dot_general(rhs1, lhs1, (((0,), (0,)), ((), ())),
                           preferred_element_type=jnp.float32) +
           lax.dot_general(rhs2, lhs2, (((0,), (0,)), ((), ())),
                           preferred_element_type=jnp.float32))
    wrt_ref[...] = wrt


def _router(x, Wg):
    return pl.pallas_call(
        _router_body,
        out_shape=(jax.ShapeDtypeStruct((2, M), jnp.int32),
                   jax.ShapeDtypeStruct((C, E + 1), jnp.float32),
                   jax.ShapeDtypeStruct((M, D // 2), jnp.int32)),
    )(x, Wg)


def _dispatch_body(x_hbm, ridx_hbm, xg_hbm, idx0, idx1, xbuf, sem0, sem1):
    wid = lax.axis_index("s") * 2 + lax.axis_index("c")
    for c in range(TPW // CH):
        base = wid * TPW + c * CH
        pltpu.sync_copy(ridx_hbm.at[pl.ds(base, CH)], idx0)
        pltpu.sync_copy(ridx_hbm.at[pl.ds(M + base, CH)], idx1)
        pltpu.sync_copy(x_hbm.at[pl.ds(base, CH)], xbuf)
        cp0 = pltpu.make_async_copy(xbuf, xg_hbm.at[idx0], sem0)
        cp1 = pltpu.make_async_copy(xbuf, xg_hbm.at[idx1], sem1)
        cp0.start()
        cp1.start()
        cp0.wait()
        cp1.wait()


@functools.cache
def _dispatch():
    return pl.kernel(
        _dispatch_body,
        out_type=jax.ShapeDtypeStruct((E * C + C, D // 2), jnp.int32),
        mesh=plsc.VectorSubcoreMesh(core_axis_name="c", subcore_axis_name="s"),
        scratch_types=[
            pltpu.VMEM((CH,), jnp.int32),
            pltpu.VMEM((CH,), jnp.int32),
            pltpu.VMEM((CH, D // 2), jnp.int32),
            pltpu.SemaphoreType.DMA,
            pltpu.SemaphoreType.DMA,
        ],
    )


def _ffn_body(xg_ref, wrt_ref, w1_ref, w2_ref, yw_ref):
    e = pl.program_id(0)
    lane = lax.broadcasted_iota(jnp.int32, (C, E + 1), 1)
    col = jnp.sum(jnp.where(lane == e, wrt_ref[...], 0.0), axis=1,
                  keepdims=True)
    xp = lax.bitcast_convert_type(xg_ref[...], jnp.uint32)
    lo = lax.bitcast_convert_type(
        (xp & 0xFFFF).astype(jnp.uint16), jnp.bfloat16).astype(jnp.float32)
    hi = lax.bitcast_convert_type(
        (xp >> 16).astype(jnp.uint16), jnp.bfloat16).astype(jnp.float32)
    xm = jnp.where(col > 0.0, jnp.concatenate([lo, hi], axis=1), 0.0)
    a = lax.dot_general(xm, w1_ref[0, :, :H], (((1,), (0,)), ((), ())),
                        preferred_element_type=jnp.float32)
    b = lax.dot_general(xm, w1_ref[0, :, H:], (((1,), (0,)), ((), ())),
                        preferred_element_type=jnp.float32)
    h = b * jax.nn.sigmoid(b) * a
    y = lax.dot_general(h, w2_ref[0], (((1,), (0,)), ((), ())),
                        preferred_element_type=jnp.float32)
    yw_ref[...] = y * col


def _ffn(xg, wrt, Wl1, Wl2):
    wcap = lambda e: jnp.minimum(e, E - 1)
    return pl.pallas_call(
        _ffn_body,
        grid=(E + 1,),
        in_specs=[
            pl.BlockSpec((C, D // 2), lambda e: (e, 0)),
            pl.BlockSpec((C, E + 1), lambda e: (0, 0)),
            pl.BlockSpec((1, D, 2 * H), lambda e: (wcap(e), 0, 0)),
            pl.BlockSpec((1, H, D), lambda e: (wcap(e), 0, 0)),
        ],
        out_specs=pl.BlockSpec((C, D), lambda e: (e, 0)),
        out_shape=jax.ShapeDtypeStruct((E * C + C, D), jnp.float32),
    )(xg, wrt, Wl1, Wl2)


def _combine_body(yw_hbm, ridx_hbm, out_hbm, idx0, idx1, buf0, buf1,
                  sem0, sem1):
    wid = lax.axis_index("s") * 2 + lax.axis_index("c")
    for c in range(TPW // CH):
        base = wid * TPW + c * CH
        pltpu.sync_copy(ridx_hbm.at[pl.ds(base, CH)], idx0)
        pltpu.sync_copy(ridx_hbm.at[pl.ds(M + base, CH)], idx1)
        cp0 = pltpu.make_async_copy(yw_hbm.at[idx0], buf0, sem0)
        cp1 = pltpu.make_async_copy(yw_hbm.at[idx1], buf1, sem1)
        cp0.start()
        cp1.start()
        cp0.wait()
        cp1.wait()

        def add_row(i, _):
            for j in range(D // 16):
                sl = pl.ds(j * 16, 16)
                buf0[i, sl] = buf0[i, sl] + buf1[i, sl]
            return 0

        lax.fori_loop(0, CH, add_row, 0)
        pltpu.sync_copy(buf0, out_hbm.at[pl.ds(base, CH)])


@functools.cache
def _combine():
    return pl.kernel(
        _combine_body,
        out_type=jax.ShapeDtypeStruct((M, D), jnp.float32),
        mesh=plsc.VectorSubcoreMesh(core_axis_name="c", subcore_axis_name="s"),
        scratch_types=[
            pltpu.VMEM((CH,), jnp.int32),
            pltpu.VMEM((CH,), jnp.int32),
            pltpu.VMEM((CH, D), jnp.float32),
            pltpu.VMEM((CH, D), jnp.float32),
            pltpu.SemaphoreType.DMA,
            pltpu.SemaphoreType.DMA,
        ],
    )


def kernel(x, Wg, Wl1, Wl2):
    ridx, wrt, xbf = _router(x, Wg)
    ridx_flat = ridx.reshape(2 * M)
    xg = _dispatch()(xbf, ridx_flat)
    yw = _ffn(xg, wrt, Wl1, Wl2)
    return _combine()(yw, ridx_flat)


# speedup vs baseline: 5.3239x; 1.0010x over previous
import functools

import jax
import jax.numpy as jnp
from jax import lax
from jax.experimental import pallas as pl
from jax.experimental.pallas import tpu as pltpu
from jax.experimental.pallas import tpu_sc as plsc

D = 768
H = 1024
E = 64
M = 2048
C = 128
HC = 1024
NHC = H // HC
NW = 32
TPW = M // NW
CH = 32
DUMP = E * C


def _router_body(x_ref, wg_ref, ridx_ref, wrt_ref, xp_ref):
    x = x_ref[...]
    xb = x.astype(jnp.bfloat16)
    lo = lax.bitcast_convert_type(xb[:, :D // 2], jnp.uint16).astype(jnp.uint32)
    hi = lax.bitcast_convert_type(xb[:, D // 2:], jnp.uint16).astype(jnp.uint32)
    xp_ref[...] = lax.bitcast_convert_type(lo | (hi << 16), jnp.int32)
    logits = lax.dot_general(x, wg_ref[...], (((1,), (0,)), ((), ())),
                             preferred_element_type=jnp.float32)
    it = lax.broadcasted_iota(jnp.int32, (M, E), 1)
    m1 = jnp.max(logits, axis=1, keepdims=True)
    i1 = jnp.min(jnp.where(logits == m1, it, E), axis=1, keepdims=True)
    l2 = jnp.where(it == i1, -jnp.inf, logits)
    m2 = jnp.max(l2, axis=1, keepdims=True)
    i2 = jnp.min(jnp.where(l2 == m2, it, E), axis=1, keepdims=True)
    w2 = jax.nn.sigmoid(m2 - m1)
    w1 = 1.0 - w2

    one = jnp.float32(1.0)
    zero = jnp.float32(0.0)
    cnt = jnp.where(it == i1, one, zero) + jnp.where(it == i2, one, zero)

    B = 256
    rio = lax.broadcasted_iota(jnp.int32, (B, B), 0)
    cio = lax.broadcasted_iota(jnp.int32, (B, B), 1)
    tri = jnp.where(rio > cio, one, zero)
    carry = jnp.zeros((1, E), jnp.float32)
    blocks = []
    for b in range(M // B):
        blk = lax.slice(cnt, (b * B, 0), ((b + 1) * B, E))
        blocks.append(
            lax.dot_general(tri, blk, (((1,), (0,)), ((), ())),
                            preferred_element_type=jnp.float32) + carry)
        carry = carry + jnp.sum(blk, axis=0, keepdims=True)
    csum = jnp.concatenate(blocks, axis=0)

    s1 = jnp.sum(jnp.where(it == i1, csum, zero), axis=1).astype(jnp.int32)
    s2 = jnp.sum(jnp.where(it == i2, csum, zero), axis=1).astype(jnp.int32)
    e1 = i1[:, 0]
    e2 = i2[:, 0]
    r1 = jnp.where(s1 < C, e1 * C + s1, DUMP)
    r2 = jnp.where(s2 < C, e2 * C + s2, DUMP)
    ridx_ref[...] = jnp.concatenate(
        [r1.reshape(1, M), r2.reshape(1, M)], axis=0)

    itE = lax.broadcasted_iota(jnp.int32, (M, E + 1), 1)
    itC = lax.broadcasted_iota(jnp.int32, (M, C), 1)
    lhs1 = jnp.where(itE == i1, w1, zero)
    lhs2 = jnp.where(itE == i2, w2, zero)
    rhs1 = jnp.where(itC == s1[:, None], one, zero)
    rhs2 = jnp.where(itC == s2[:, None], one, zero)
    wrt = (lax.dot_general(rhs1, lhs1, (((0,), (0,)), ((), ())),
                           preferred_element_type=jnp.float32) +
           lax.dot_general(rhs2, lhs2, (((0,), (0,)), ((), ())),
                           preferred_element_type=jnp.float32))
    wrt_ref[...] = wrt


def _router(x, Wg):
    return pl.pallas_call(
        _router_body,
        out_shape=(jax.ShapeDtypeStruct((2, M), jnp.int32),
                   jax.ShapeDtypeStruct((C, E + 1), jnp.float32),
                   jax.ShapeDtypeStruct((M, D // 2), jnp.int32)),
    )(x, Wg)


def _dispatch_body(x_hbm, ridx_hbm, xg_hbm, idx0, idx1, xbuf, sem0, sem1):
    wid = lax.axis_index("s") * 2 + lax.axis_index("c")
    for c in range(TPW // CH):
        base = wid * TPW + c * CH
        pltpu.sync_copy(ridx_hbm.at[pl.ds(base, CH)], idx0)
        pltpu.sync_copy(ridx_hbm.at[pl.ds(M + base, CH)], idx1)
        pltpu.sync_copy(x_hbm.at[pl.ds(base, CH)], xbuf)
        cp0 = pltpu.make_async_copy(xbuf, xg_hbm.at[idx0], sem0)
        cp1 = pltpu.make_async_copy(xbuf, xg_hbm.at[idx1], sem1)
        cp0.start()
        cp1.start()
        cp0.wait()
        cp1.wait()


@functools.cache
def _dispatch():
    return pl.kernel(
        _dispatch_body,
        out_type=jax.ShapeDtypeStruct((E * C + C, D // 2), jnp.int32),
        mesh=plsc.VectorSubcoreMesh(core_axis_name="c", subcore_axis_name="s"),
        scratch_types=[
            pltpu.VMEM((CH,), jnp.int32),
            pltpu.VMEM((CH,), jnp.int32),
            pltpu.VMEM((CH, D // 2), jnp.int32),
            pltpu.SemaphoreType.DMA,
            pltpu.SemaphoreType.DMA,
        ],
    )


def _ffn_body(xg_ref, wrt_ref, w1_ref, w2_ref, yw_ref):
    e = pl.program_id(0)
    lane = lax.broadcasted_iota(jnp.int32, (C, E + 1), 1)
    col = jnp.sum(jnp.where(lane == e, wrt_ref[...], 0.0), axis=1,
                  keepdims=True)
    xp = lax.bitcast_convert_type(xg_ref[...], jnp.uint32)
    lo = lax.bitcast_convert_type(
        (xp & 0xFFFF).astype(jnp.uint16), jnp.bfloat16).astype(jnp.float32)
    hi = lax.bitcast_convert_type(
        (xp >> 16).astype(jnp.uint16), jnp.bfloat16).astype(jnp.float32)
    xm = jnp.where(col > 0.0, jnp.concatenate([lo, hi], axis=1), 0.0)
    a = lax.dot_general(xm, w1_ref[0, :, :H], (((1,), (0,)), ((), ())),
                        preferred_element_type=jnp.float32)
    b = lax.dot_general(xm, w1_ref[0, :, H:], (((1,), (0,)), ((), ())),
                        preferred_element_type=jnp.float32)
    h = b * jax.nn.sigmoid(b) * a
    y = lax.dot_general(h, w2_ref[0], (((1,), (0,)), ((), ())),
                        preferred_element_type=jnp.float32)
    yb = (y * col).astype(jnp.bfloat16)
    ylo = lax.bitcast_convert_type(yb[:, :D // 2], jnp.uint16).astype(jnp.uint32)
    yhi = lax.bitcast_convert_type(yb[:, D // 2:], jnp.uint16).astype(jnp.uint32)
    yw_ref[...] = lax.bitcast_convert_type(ylo | (yhi << 16), jnp.int32)


def _ffn(xg, wrt, Wl1, Wl2):
    wcap = lambda e: jnp.minimum(e, E - 1)
    return pl.pallas_call(
        _ffn_body,
        grid=(E + 1,),
        in_specs=[
            pl.BlockSpec((C, D // 2), lambda e: (e, 0)),
            pl.BlockSpec((C, E + 1), lambda e: (0, 0)),
            pl.BlockSpec((1, D, 2 * H), lambda e: (wcap(e), 0, 0)),
            pl.BlockSpec((1, H, D), lambda e: (wcap(e), 0, 0)),
        ],
        out_specs=pl.BlockSpec((C, D // 2), lambda e: (e, 0)),
        out_shape=jax.ShapeDtypeStruct((E * C + C, D // 2), jnp.int32),
    )(xg, wrt, Wl1, Wl2)


def _combine_body(yw_hbm, ridx_hbm, out_hbm, idx0, idx1, buf0, buf1, obuf,
                  sem0, sem1):
    wid = lax.axis_index("s") * 2 + lax.axis_index("c")
    for c in range(TPW // CH):
        base = wid * TPW + c * CH
        pltpu.sync_copy(ridx_hbm.at[pl.ds(base, CH)], idx0)
        pltpu.sync_copy(ridx_hbm.at[pl.ds(M + base, CH)], idx1)
        cp0 = pltpu.make_async_copy(yw_hbm.at[idx0], buf0, sem0)
        cp1 = pltpu.make_async_copy(yw_hbm.at[idx1], buf1, sem1)
        cp0.start()
        cp1.start()
        cp0.wait()
        cp1.wait()

        def add_row(i, _):
            himask = jnp.int32(-65536)
            for j in range(D // 2 // 16):
                sl = pl.ds(j * 16, 16)
                v0 = buf0[i, sl]
                v1 = buf1[i, sl]
                lo = (lax.bitcast_convert_type(v0 << 16, jnp.float32) +
                      lax.bitcast_convert_type(v1 << 16, jnp.float32))
                hi = (lax.bitcast_convert_type(v0 & himask, jnp.float32) +
                      lax.bitcast_convert_type(v1 & himask, jnp.float32))
                obuf[i, sl] = lo
                obuf[i, pl.ds(D // 2 + j * 16, 16)] = hi
            return 0

        lax.fori_loop(0, CH, add_row, 0)
        pltpu.sync_copy(obuf, out_hbm.at[pl.ds(base, CH)])


@functools.cache
def _combine():
    return pl.kernel(
        _combine_body,
        out_type=jax.ShapeDtypeStruct((M, D), jnp.float32),
        mesh=plsc.VectorSubcoreMesh(core_axis_name="c", subcore_axis_name="s"),
        scratch_types=[
            pltpu.VMEM((CH,), jnp.int32),
            pltpu.VMEM((CH,), jnp.int32),
            pltpu.VMEM((CH, D // 2), jnp.int32),
            pltpu.VMEM((CH, D // 2), jnp.int32),
            pltpu.VMEM((CH, D), jnp.float32),
            pltpu.SemaphoreType.DMA,
            pltpu.SemaphoreType.DMA,
        ],
    )


def kernel(x, Wg, Wl1, Wl2):
    ridx, wrt, xbf = _router(x, Wg)
    ridx_flat = ridx.reshape(2 * M)
    xg = _dispatch()(xbf, ridx_flat)
    yw = _ffn(xg, wrt, Wl1, Wl2)
    return _combine()(yw, ridx_flat)


# speedup vs baseline: 5.4262x; 1.0192x over previous
import functools

import jax
import jax.numpy as jnp
from jax import lax
from jax.experimental import pallas as pl
from jax.experimental.pallas import tpu as pltpu
from jax.experimental.pallas import tpu_sc as plsc

D = 768
H = 1024
E = 64
M = 2048
C = 128
HC = 1024
NHC = H // HC
NW = 32
TPW = M // NW
CH = 32
DUMP = E * C


def _router_body(x_ref, wg_ref, ridx_ref, wrt_ref, xp_ref):
    x = x_ref[...]
    xb = x.astype(jnp.bfloat16)
    lo = lax.bitcast_convert_type(xb[:, :D // 2], jnp.uint16).astype(jnp.uint32)
    hi = lax.bitcast_convert_type(xb[:, D // 2:], jnp.uint16).astype(jnp.uint32)
    xp_ref[...] = lax.bitcast_convert_type(lo | (hi << 16), jnp.int32)
    logits = lax.dot_general(x, wg_ref[...], (((1,), (0,)), ((), ())),
                             preferred_element_type=jnp.float32)
    it = lax.broadcasted_iota(jnp.int32, (M, E), 1)
    m1 = jnp.max(logits, axis=1, keepdims=True)
    i1 = jnp.min(jnp.where(logits == m1, it, E), axis=1, keepdims=True)
    l2 = jnp.where(it == i1, -jnp.inf, logits)
    m2 = jnp.max(l2, axis=1, keepdims=True)
    i2 = jnp.min(jnp.where(l2 == m2, it, E), axis=1, keepdims=True)
    w2 = jax.nn.sigmoid(m2 - m1)
    w1 = 1.0 - w2

    one = jnp.float32(1.0)
    zero = jnp.float32(0.0)
    cnt = jnp.where(it == i1, one, zero) + jnp.where(it == i2, one, zero)

    B = 256
    rio = lax.broadcasted_iota(jnp.int32, (B, B), 0)
    cio = lax.broadcasted_iota(jnp.int32, (B, B), 1)
    tri = jnp.where(rio > cio, one, zero)
    carry = jnp.zeros((1, E), jnp.float32)
    blocks = []
    for b in range(M // B):
        blk = lax.slice(cnt, (b * B, 0), ((b + 1) * B, E))
        blocks.append(
            lax.dot_general(tri, blk, (((1,), (0,)), ((), ())),
                            preferred_element_type=jnp.float32) + carry)
        carry = carry + jnp.sum(blk, axis=0, keepdims=True)
    csum = jnp.concatenate(blocks, axis=0)

    s1 = jnp.sum(jnp.where(it == i1, csum, zero), axis=1).astype(jnp.int32)
    s2 = jnp.sum(jnp.where(it == i2, csum, zero), axis=1).astype(jnp.int32)
    e1 = i1[:, 0]
    e2 = i2[:, 0]
    r1 = jnp.where(s1 < C, e1 * C + s1, DUMP)
    r2 = jnp.where(s2 < C, e2 * C + s2, DUMP)
    ridx_ref[...] = jnp.concatenate(
        [r1.reshape(1, M), r2.reshape(1, M)], axis=0)

    itE = lax.broadcasted_iota(jnp.int32, (M, E + 1), 1)
    itC = lax.broadcasted_iota(jnp.int32, (M, C), 1)
    lhs1 = jnp.where(itE == i1, w1, zero)
    lhs2 = jnp.where(itE == i2, w2, zero)
    rhs1 = jnp.where(itC == s1[:, None], one, zero)
    rhs2 = jnp.where(itC == s2[:, None], one, zero)
    wrt = (lax.dot_general(rhs1, lhs1, (((0,), (0,)), ((), ())),
                           preferred_element_type=jnp.float32) +
           lax.dot_general(rhs2, lhs2, (((0,), (0,)), ((), ())),
                           preferred_element_type=jnp.float32))
    wrt_ref[...] = wrt


def _router(x, Wg):
    return pl.pallas_call(
        _router_body,
        out_shape=(jax.ShapeDtypeStruct((2, M), jnp.int32),
                   jax.ShapeDtypeStruct((C, E + 1), jnp.float32),
                   jax.ShapeDtypeStruct((M, D // 2), jnp.int32)),
    )(x, Wg)


def _dispatch_body(x_hbm, ridx_hbm, xg_hbm, idx0a, idx1a, idx0b, idx1b,
                   xba, xbb, sem):
    wid = lax.axis_index("s") * 2 + lax.axis_index("c")
    cps = []
    for c, (i0, i1, xb) in enumerate(((idx0a, idx1a, xba),
                                      (idx0b, idx1b, xbb))):
        base = wid * TPW + c * CH
        pltpu.sync_copy(ridx_hbm.at[pl.ds(base, CH)], i0)
        pltpu.sync_copy(ridx_hbm.at[pl.ds(M + base, CH)], i1)
        pltpu.sync_copy(x_hbm.at[pl.ds(base, CH)], xb)
        cp0 = pltpu.make_async_copy(xb, xg_hbm.at[i0], sem)
        cp1 = pltpu.make_async_copy(xb, xg_hbm.at[i1], sem)
        cp0.start()
        cp1.start()
        cps += [cp0, cp1]
    for cp in cps:
        cp.wait()


@functools.cache
def _dispatch():
    return pl.kernel(
        _dispatch_body,
        out_type=jax.ShapeDtypeStruct((E * C + C, D // 2), jnp.int32),
        mesh=plsc.VectorSubcoreMesh(core_axis_name="c", subcore_axis_name="s"),
        scratch_types=[
            pltpu.VMEM((CH,), jnp.int32),
            pltpu.VMEM((CH,), jnp.int32),
            pltpu.VMEM((CH,), jnp.int32),
            pltpu.VMEM((CH,), jnp.int32),
            pltpu.VMEM((CH, D // 2), jnp.int32),
            pltpu.VMEM((CH, D // 2), jnp.int32),
            pltpu.SemaphoreType.DMA,
        ],
    )


def _ffn_body(xg_ref, wrt_ref, w1_ref, w2_ref, yw_ref):
    e = pl.program_id(0)
    lane = lax.broadcasted_iota(jnp.int32, (C, E + 1), 1)
    col = jnp.sum(jnp.where(lane == e, wrt_ref[...], 0.0), axis=1,
                  keepdims=True)
    xp = lax.bitcast_convert_type(xg_ref[...], jnp.uint32)
    lo = lax.bitcast_convert_type(
        (xp & 0xFFFF).astype(jnp.uint16), jnp.bfloat16).astype(jnp.float32)
    hi = lax.bitcast_convert_type(
        (xp >> 16).astype(jnp.uint16), jnp.bfloat16).astype(jnp.float32)
    xm = jnp.where(col > 0.0, jnp.concatenate([lo, hi], axis=1), 0.0)
    a = lax.dot_general(xm, w1_ref[0, :, :H], (((1,), (0,)), ((), ())),
                        preferred_element_type=jnp.float32)
    b = lax.dot_general(xm, w1_ref[0, :, H:], (((1,), (0,)), ((), ())),
                        preferred_element_type=jnp.float32)
    h = b * jax.nn.sigmoid(b) * a
    y = lax.dot_general(h, w2_ref[0], (((1,), (0,)), ((), ())),
                        preferred_element_type=jnp.float32)
    yb = (y * col).astype(jnp.bfloat16)
    ylo = lax.bitcast_convert_type(yb[:, :D // 2], jnp.uint16).astype(jnp.uint32)
    yhi = lax.bitcast_convert_type(yb[:, D // 2:], jnp.uint16).astype(jnp.uint32)
    yw_ref[...] = lax.bitcast_convert_type(ylo | (yhi << 16), jnp.int32)


def _ffn(xg, wrt, Wl1, Wl2):
    wcap = lambda e: jnp.minimum(e, E - 1)
    return pl.pallas_call(
        _ffn_body,
        grid=(E + 1,),
        in_specs=[
            pl.BlockSpec((C, D // 2), lambda e: (e, 0)),
            pl.BlockSpec((C, E + 1), lambda e: (0, 0)),
            pl.BlockSpec((1, D, 2 * H), lambda e: (wcap(e), 0, 0)),
            pl.BlockSpec((1, H, D), lambda e: (wcap(e), 0, 0)),
        ],
        out_specs=pl.BlockSpec((C, D // 2), lambda e: (e, 0)),
        out_shape=jax.ShapeDtypeStruct((E * C + C, D // 2), jnp.int32),
    )(xg, wrt, Wl1, Wl2)


def _add_chunk(buf0, buf1, obuf):
    himask = jnp.int32(-65536)

    @plsc.parallel_loop(0, CH, 1, unroll=2)
    def _(i):
        for j in range(D // 2 // 16):
            sl = pl.ds(j * 16, 16)
            v0 = buf0[i, sl]
            v1 = buf1[i, sl]
            lo = (lax.bitcast_convert_type(v0 << 16, jnp.float32) +
                  lax.bitcast_convert_type(v1 << 16, jnp.float32))
            hi = (lax.bitcast_convert_type(v0 & himask, jnp.float32) +
                  lax.bitcast_convert_type(v1 & himask, jnp.float32))
            obuf[i, sl] = lo
            obuf[i, pl.ds(D // 2 + j * 16, 16)] = hi


def _combine_body(yw_hbm, ridx_hbm, out_hbm, idx0a, idx1a, idx0b, idx1b,
                  b0a, b1a, b0b, b1b, oba, obb, sema, semb, semo):
    wid = lax.axis_index("s") * 2 + lax.axis_index("c")
    base0 = wid * TPW
    base1 = base0 + CH
    gathers = []
    for base, i0, i1, b0, b1, sem in ((base0, idx0a, idx1a, b0a, b1a, sema),
                                      (base1, idx0b, idx1b, b0b, b1b, semb)):
        pltpu.sync_copy(ridx_hbm.at[pl.ds(base, CH)], i0)
        pltpu.sync_copy(ridx_hbm.at[pl.ds(M + base, CH)], i1)
        cp0 = pltpu.make_async_copy(yw_hbm.at[i0], b0, sem)
        cp1 = pltpu.make_async_copy(yw_hbm.at[i1], b1, sem)
        cp0.start()
        cp1.start()
        gathers += [cp0, cp1]
    gathers[0].wait()
    gathers[1].wait()
    _add_chunk(b0a, b1a, oba)
    co0 = pltpu.make_async_copy(oba, out_hbm.at[pl.ds(base0, CH)], semo)
    co0.start()
    gathers[2].wait()
    gathers[3].wait()
    _add_chunk(b0b, b1b, obb)
    co1 = pltpu.make_async_copy(obb, out_hbm.at[pl.ds(base1, CH)], semo)
    co1.start()
    co0.wait()
    co1.wait()


@functools.cache
def _combine():
    return pl.kernel(
        _combine_body,
        out_type=jax.ShapeDtypeStruct((M, D), jnp.float32),
        mesh=plsc.VectorSubcoreMesh(core_axis_name="c", subcore_axis_name="s"),
        scratch_types=[
            pltpu.VMEM((CH,), jnp.int32),
            pltpu.VMEM((CH,), jnp.int32),
            pltpu.VMEM((CH,), jnp.int32),
            pltpu.VMEM((CH,), jnp.int32),
            pltpu.VMEM((CH, D // 2), jnp.int32),
            pltpu.VMEM((CH, D // 2), jnp.int32),
            pltpu.VMEM((CH, D // 2), jnp.int32),
            pltpu.VMEM((CH, D // 2), jnp.int32),
            pltpu.VMEM((CH, D), jnp.float32),
            pltpu.VMEM((CH, D), jnp.float32),
            pltpu.SemaphoreType.DMA,
            pltpu.SemaphoreType.DMA,
            pltpu.SemaphoreType.DMA,
        ],
    )


def kernel(x, Wg, Wl1, Wl2):
    ridx, wrt, xbf = _router(x, Wg)
    ridx_flat = ridx.reshape(2 * M)
    xg = _dispatch()(xbf, ridx_flat)
    yw = _ffn(xg, wrt, Wl1, Wl2)
    return _combine()(yw, ridx_flat)


# speedup vs baseline: 5.5233x; 1.0179x over previous
import functools

import jax
import jax.numpy as jnp
from jax import lax
from jax.experimental import pallas as pl
from jax.experimental.pallas import tpu as pltpu
from jax.experimental.pallas import tpu_sc as plsc

D = 768
H = 1024
E = 64
M = 2048
C = 128
HC = 1024
NHC = H // HC
NW = 32
TPW = M // NW
CH = 32


def _router_body(x_ref, wg_ref, ridx_ref, wrt_ref, xp_ref):
    x = x_ref[...]
    xb = x.astype(jnp.bfloat16)
    lo = lax.bitcast_convert_type(xb[:, :D // 2], jnp.uint16).astype(jnp.uint32)
    hi = lax.bitcast_convert_type(xb[:, D // 2:], jnp.uint16).astype(jnp.uint32)
    xp_ref[...] = lax.bitcast_convert_type(lo | (hi << 16), jnp.int32)
    logits = lax.dot_general(x, wg_ref[...], (((1,), (0,)), ((), ())),
                             preferred_element_type=jnp.float32)
    it = lax.broadcasted_iota(jnp.int32, (M, E), 1)
    m1 = jnp.max(logits, axis=1, keepdims=True)
    i1 = jnp.min(jnp.where(logits == m1, it, E), axis=1, keepdims=True)
    l2 = jnp.where(it == i1, -jnp.inf, logits)
    m2 = jnp.max(l2, axis=1, keepdims=True)
    i2 = jnp.min(jnp.where(l2 == m2, it, E), axis=1, keepdims=True)
    w2 = jax.nn.sigmoid(m2 - m1)
    w1 = 1.0 - w2

    one = jnp.float32(1.0)
    zero = jnp.float32(0.0)
    cnt = jnp.where(it == i1, one, zero) + jnp.where(it == i2, one, zero)

    B = 256
    rio = lax.broadcasted_iota(jnp.int32, (B, B), 0)
    cio = lax.broadcasted_iota(jnp.int32, (B, B), 1)
    tri = jnp.where(rio > cio, one, zero)
    carry = jnp.zeros((1, E), jnp.float32)
    blocks = []
    for b in range(M // B):
        blk = lax.slice(cnt, (b * B, 0), ((b + 1) * B, E))
        blocks.append(
            lax.dot_general(tri, blk, (((1,), (0,)), ((), ())),
                            preferred_element_type=jnp.float32) + carry)
        carry = carry + jnp.sum(blk, axis=0, keepdims=True)
    csum = jnp.concatenate(blocks, axis=0)

    s1 = jnp.sum(jnp.where(it == i1, csum, zero), axis=1).astype(jnp.int32)
    s2 = jnp.sum(jnp.where(it == i2, csum, zero), axis=1).astype(jnp.int32)
    e1 = i1[:, 0]
    e2 = i2[:, 0]
    cmin = jnp.min(carry, axis=1, keepdims=True)
    eit = lax.broadcasted_iota(jnp.int32, (1, E), 1)
    emin = jnp.min(jnp.where(carry == cmin, eit, E), axis=1)
    free_row = emin[0] * C + cmin[0, 0].astype(jnp.int32)
    r1 = jnp.where(s1 < C, e1 * C + s1, free_row)
    r2 = jnp.where(s2 < C, e2 * C + s2, free_row)
    ridx_ref[...] = jnp.concatenate(
        [r1.reshape(1, M), r2.reshape(1, M)], axis=0)

    itE = lax.broadcasted_iota(jnp.int32, (M, E + 1), 1)
    itC = lax.broadcasted_iota(jnp.int32, (M, C), 1)
    lhs1 = jnp.where(itE == i1, w1, zero)
    lhs2 = jnp.where(itE == i2, w2, zero)
    rhs1 = jnp.where(itC == s1[:, None], one, zero)
    rhs2 = jnp.where(itC == s2[:, None], one, zero)
    wrt = (lax.dot_general(rhs1, lhs1, (((0,), (0,)), ((), ())),
                           preferred_element_type=jnp.float32) +
           lax.dot_general(rhs2, lhs2, (((0,), (0,)), ((), ())),
                           preferred_element_type=jnp.float32))
    wrt_ref[...] = wrt


def _router(x, Wg):
    return pl.pallas_call(
        _router_body,
        out_shape=(jax.ShapeDtypeStruct((2, M), jnp.int32),
                   jax.ShapeDtypeStruct((C, E + 1), jnp.float32),
                   jax.ShapeDtypeStruct((M, D // 2), jnp.int32)),
    )(x, Wg)


def _dispatch_body(x_hbm, ridx_hbm, xg_hbm, idx0a, idx1a, idx0b, idx1b,
                   xba, xbb, sem):
    wid = lax.axis_index("s") * 2 + lax.axis_index("c")
    cps = []
    for c, (i0, i1, xb) in enumerate(((idx0a, idx1a, xba),
                                      (idx0b, idx1b, xbb))):
        base = wid * TPW + c * CH
        pltpu.sync_copy(ridx_hbm.at[pl.ds(base, CH)], i0)
        pltpu.sync_copy(ridx_hbm.at[pl.ds(M + base, CH)], i1)
        pltpu.sync_copy(x_hbm.at[pl.ds(base, CH)], xb)
        cp0 = pltpu.make_async_copy(xb, xg_hbm.at[i0], sem)
        cp1 = pltpu.make_async_copy(xb, xg_hbm.at[i1], sem)
        cp0.start()
        cp1.start()
        cps += [cp0, cp1]
    for cp in cps:
        cp.wait()


@functools.cache
def _dispatch():
    return pl.kernel(
        _dispatch_body,
        out_type=jax.ShapeDtypeStruct((E * C, D // 2), jnp.int32),
        mesh=plsc.VectorSubcoreMesh(core_axis_name="c", subcore_axis_name="s"),
        scratch_types=[
            pltpu.VMEM((CH,), jnp.int32),
            pltpu.VMEM((CH,), jnp.int32),
            pltpu.VMEM((CH,), jnp.int32),
            pltpu.VMEM((CH,), jnp.int32),
            pltpu.VMEM((CH, D // 2), jnp.int32),
            pltpu.VMEM((CH, D // 2), jnp.int32),
            pltpu.SemaphoreType.DMA,
        ],
    )


def _ffn_body(xg_ref, wrt_ref, w1_ref, w2_ref, yw_ref):
    e = pl.program_id(0)
    lane = lax.broadcasted_iota(jnp.int32, (C, E + 1), 1)
    col = jnp.sum(jnp.where(lane == e, wrt_ref[...], 0.0), axis=1,
                  keepdims=True)
    xp = lax.bitcast_convert_type(xg_ref[...], jnp.uint32)
    lo = lax.bitcast_convert_type(
        (xp & 0xFFFF).astype(jnp.uint16), jnp.bfloat16).astype(jnp.float32)
    hi = lax.bitcast_convert_type(
        (xp >> 16).astype(jnp.uint16), jnp.bfloat16).astype(jnp.float32)
    xm = jnp.where(col > 0.0, jnp.concatenate([lo, hi], axis=1), 0.0)
    a = lax.dot_general(xm, w1_ref[0, :, :H], (((1,), (0,)), ((), ())),
                        preferred_element_type=jnp.float32)
    b = lax.dot_general(xm, w1_ref[0, :, H:], (((1,), (0,)), ((), ())),
                        preferred_element_type=jnp.float32)
    h = b * jax.nn.sigmoid(b) * a
    y = lax.dot_general(h, w2_ref[0], (((1,), (0,)), ((), ())),
                        preferred_element_type=jnp.float32)
    yb = (y * col).astype(jnp.bfloat16)
    ylo = lax.bitcast_convert_type(yb[:, :D // 2], jnp.uint16).astype(jnp.uint32)
    yhi = lax.bitcast_convert_type(yb[:, D // 2:], jnp.uint16).astype(jnp.uint32)
    yw_ref[...] = lax.bitcast_convert_type(ylo | (yhi << 16), jnp.int32)


def _ffn(xg, wrt, Wl1, Wl2):
    return pl.pallas_call(
        _ffn_body,
        grid=(E,),
        in_specs=[
            pl.BlockSpec((C, D // 2), lambda e: (e, 0)),
            pl.BlockSpec((C, E + 1), lambda e: (0, 0)),
            pl.BlockSpec((1, D, 2 * H), lambda e: (e, 0, 0)),
            pl.BlockSpec((1, H, D), lambda e: (e, 0, 0)),
        ],
        out_specs=pl.BlockSpec((C, D // 2), lambda e: (e, 0)),
        out_shape=jax.ShapeDtypeStruct((E * C, D // 2), jnp.int32),
    )(xg, wrt, Wl1, Wl2)


def _add_chunk(buf0, buf1, obuf):
    himask = jnp.int32(-65536)

    @plsc.parallel_loop(0, CH, 1, unroll=2)
    def _(i):
        for j in range(D // 2 // 16):
            sl = pl.ds(j * 16, 16)
            v0 = buf0[i, sl]
            v1 = buf1[i, sl]
            lo = (lax.bitcast_convert_type(v0 << 16, jnp.float32) +
                  lax.bitcast_convert_type(v1 << 16, jnp.float32))
            hi = (lax.bitcast_convert_type(v0 & himask, jnp.float32) +
                  lax.bitcast_convert_type(v1 & himask, jnp.float32))
            obuf[i, sl] = lo
            obuf[i, pl.ds(D // 2 + j * 16, 16)] = hi


def _combine_body(yw_hbm, ridx_hbm, out_hbm, idx0a, idx1a, idx0b, idx1b,
                  b0a, b1a, b0b, b1b, oba, obb, sema, semb, semo):
    wid = lax.axis_index("s") * 2 + lax.axis_index("c")
    base0 = wid * TPW
    base1 = base0 + CH
    gathers = []
    for base, i0, i1, b0, b1, sem in ((base0, idx0a, idx1a, b0a, b1a, sema),
                                      (base1, idx0b, idx1b, b0b, b1b, semb)):
        pltpu.sync_copy(ridx_hbm.at[pl.ds(base, CH)], i0)
        pltpu.sync_copy(ridx_hbm.at[pl.ds(M + base, CH)], i1)
        cp0 = pltpu.make_async_copy(yw_hbm.at[i0], b0, sem)
        cp1 = pltpu.make_async_copy(yw_hbm.at[i1], b1, sem)
        cp0.start()
        cp1.start()
        gathers += [cp0, cp1]
    gathers[0].wait()
    gathers[1].wait()
    _add_chunk(b0a, b1a, oba)
    co0 = pltpu.make_async_copy(oba, out_hbm.at[pl.ds(base0, CH)], semo)
    co0.start()
    gathers[2].wait()
    gathers[3].wait()
    _add_chunk(b0b, b1b, obb)
    co1 = pltpu.make_async_copy(obb, out_hbm.at[pl.ds(base1, CH)], semo)
    co1.start()
    co0.wait()
    co1.wait()


@functools.cache
def _combine():
    return pl.kernel(
        _combine_body,
        out_type=jax.ShapeDtypeStruct((M, D), jnp.float32),
        mesh=plsc.VectorSubcoreMesh(core_axis_name="c", subcore_axis_name="s"),
        scratch_types=[
            pltpu.VMEM((CH,), jnp.int32),
            pltpu.VMEM((CH,), jnp.int32),
            pltpu.VMEM((CH,), jnp.int32),
            pltpu.VMEM((CH,), jnp.int32),
            pltpu.VMEM((CH, D // 2), jnp.int32),
            pltpu.VMEM((CH, D // 2), jnp.int32),
            pltpu.VMEM((CH, D // 2), jnp.int32),
            pltpu.VMEM((CH, D // 2), jnp.int32),
            pltpu.VMEM((CH, D), jnp.float32),
            pltpu.VMEM((CH, D), jnp.float32),
            pltpu.SemaphoreType.DMA,
            pltpu.SemaphoreType.DMA,
            pltpu.SemaphoreType.DMA,
        ],
    )


def kernel(x, Wg, Wl1, Wl2):
    ridx, wrt, xbf = _router(x, Wg)
    ridx_flat = ridx.reshape(2 * M)
    xg = _dispatch()(xbf, ridx_flat)
    yw = _ffn(xg, wrt, Wl1, Wl2)
    return _combine()(yw, ridx_flat)


# speedup vs baseline: 5.5284x; 1.0009x over previous
import functools

import jax
import jax.numpy as jnp
from jax import lax
from jax.experimental import pallas as pl
from jax.experimental.pallas import tpu as pltpu
from jax.experimental.pallas import tpu_sc as plsc

D = 768
H = 1024
E = 64
M = 2048
C = 128
HC = 1024
NHC = H // HC
NW = 32
TPW = M // NW
CH = 32


def _router_body(x_ref, wg_ref, ridx_ref, wrt_ref, xp_ref):
    x = x_ref[...]
    xb = x.astype(jnp.bfloat16)
    lo = lax.bitcast_convert_type(xb[:, :D // 2], jnp.uint16).astype(jnp.uint32)
    hi = lax.bitcast_convert_type(xb[:, D // 2:], jnp.uint16).astype(jnp.uint32)
    xp_ref[...] = lax.bitcast_convert_type(lo | (hi << 16), jnp.int32)
    logits = lax.dot_general(x, wg_ref[...], (((1,), (0,)), ((), ())),
                             preferred_element_type=jnp.float32)
    it = lax.broadcasted_iota(jnp.int32, (M, E), 1)
    m1 = jnp.max(logits, axis=1, keepdims=True)
    i1 = jnp.min(jnp.where(logits == m1, it, E), axis=1, keepdims=True)
    l2 = jnp.where(it == i1, -jnp.inf, logits)
    m2 = jnp.max(l2, axis=1, keepdims=True)
    i2 = jnp.min(jnp.where(l2 == m2, it, E), axis=1, keepdims=True)
    w2 = jax.nn.sigmoid(m2 - m1)
    w1 = 1.0 - w2

    one = jnp.float32(1.0)
    zero = jnp.float32(0.0)
    cnt = jnp.where(it == i1, one, zero) + jnp.where(it == i2, one, zero)

    B = 256
    rio = lax.broadcasted_iota(jnp.int32, (B, B), 0)
    cio = lax.broadcasted_iota(jnp.int32, (B, B), 1)
    tri = jnp.where(rio > cio, one, zero)
    carry = jnp.zeros((1, E), jnp.float32)
    blocks = []
    for b in range(M // B):
        blk = lax.slice(cnt, (b * B, 0), ((b + 1) * B, E))
        blocks.append(
            lax.dot_general(tri, blk, (((1,), (0,)), ((), ())),
                            preferred_element_type=jnp.float32) + carry)
        carry = carry + jnp.sum(blk, axis=0, keepdims=True)
    csum = jnp.concatenate(blocks, axis=0)

    s1 = jnp.sum(jnp.where(it == i1, csum, zero), axis=1).astype(jnp.int32)
    s2 = jnp.sum(jnp.where(it == i2, csum, zero), axis=1).astype(jnp.int32)
    e1 = i1[:, 0]
    e2 = i2[:, 0]
    cmin = jnp.min(carry, axis=1, keepdims=True)
    eit = lax.broadcasted_iota(jnp.int32, (1, E), 1)
    emin = jnp.min(jnp.where(carry == cmin, eit, E), axis=1)
    free_row = emin[0] * C + cmin[0, 0].astype(jnp.int32)
    r1 = jnp.where(s1 < C, e1 * C + s1, free_row)
    r2 = jnp.where(s2 < C, e2 * C + s2, free_row)
    ridx_ref[...] = jnp.concatenate(
        [r1.reshape(1, M), r2.reshape(1, M)], axis=0)

    itE = lax.broadcasted_iota(jnp.int32, (M, E + 1), 1)
    itC = lax.broadcasted_iota(jnp.int32, (M, C), 1)
    lhs1 = jnp.where(itE == i1, w1, zero)
    lhs2 = jnp.where(itE == i2, w2, zero)
    rhs1 = jnp.where(itC == s1[:, None], one, zero)
    rhs2 = jnp.where(itC == s2[:, None], one, zero)
    wrt = (lax.dot_general(rhs1, lhs1, (((0,), (0,)), ((), ())),
                           preferred_element_type=jnp.float32) +
           lax.dot_general(rhs2, lhs2, (((0,), (0,)), ((), ())),
                           preferred_element_type=jnp.float32))
    wrt_ref[...] = wrt


def _router(x, Wg):
    return pl.pallas_call(
        _router_body,
        out_shape=(jax.ShapeDtypeStruct((2, M), jnp.int32),
                   jax.ShapeDtypeStruct((C, E + 1), jnp.float32),
                   jax.ShapeDtypeStruct((M, D // 2), jnp.int32)),
    )(x, Wg)


def _dispatch_body(x_hbm, ridx_hbm, xg_hbm, idx0a, idx1a, idx0b, idx1b,
                   xba, xbb, sem):
    wid = lax.axis_index("s") * 2 + lax.axis_index("c")
    cps = []
    for c, (i0, i1, xb) in enumerate(((idx0a, idx1a, xba),
                                      (idx0b, idx1b, xbb))):
        base = wid * TPW + c * CH
        pltpu.sync_copy(ridx_hbm.at[pl.ds(base, CH)], i0)
        pltpu.sync_copy(ridx_hbm.at[pl.ds(M + base, CH)], i1)
        pltpu.sync_copy(x_hbm.at[pl.ds(base, CH)], xb)
        cp0 = pltpu.make_async_copy(xb, xg_hbm.at[i0], sem)
        cp1 = pltpu.make_async_copy(xb, xg_hbm.at[i1], sem)
        cp0.start()
        cp1.start()
        cps += [cp0, cp1]
    for cp in cps:
        cp.wait()


@functools.cache
def _dispatch():
    return pl.kernel(
        _dispatch_body,
        out_type=jax.ShapeDtypeStruct((E * C, D // 2), jnp.int32),
        mesh=plsc.VectorSubcoreMesh(core_axis_name="c", subcore_axis_name="s"),
        scratch_types=[
            pltpu.VMEM((CH,), jnp.int32),
            pltpu.VMEM((CH,), jnp.int32),
            pltpu.VMEM((CH,), jnp.int32),
            pltpu.VMEM((CH,), jnp.int32),
            pltpu.VMEM((CH, D // 2), jnp.int32),
            pltpu.VMEM((CH, D // 2), jnp.int32),
            pltpu.SemaphoreType.DMA,
        ],
    )


def _ffn_body(xg_ref, wrt_ref, w1_ref, w2_ref, yw_ref):
    e = pl.program_id(0)
    lane = lax.broadcasted_iota(jnp.int32, (C, E + 1), 1)
    col = jnp.sum(jnp.where(lane == e, wrt_ref[...], 0.0), axis=1,
                  keepdims=True)
    xp = lax.bitcast_convert_type(xg_ref[...], jnp.uint32)
    lo = lax.bitcast_convert_type(
        (xp & 0xFFFF).astype(jnp.uint16), jnp.bfloat16)
    hi = lax.bitcast_convert_type(
        (xp >> 16).astype(jnp.uint16), jnp.bfloat16)
    xm = jnp.where(col > 0.0, jnp.concatenate([lo, hi], axis=1),
                   jnp.bfloat16(0.0))
    w1 = w1_ref[0].astype(jnp.bfloat16)
    a = lax.dot_general(xm, w1[:, :H], (((1,), (0,)), ((), ())),
                        preferred_element_type=jnp.float32)
    b = lax.dot_general(xm, w1[:, H:], (((1,), (0,)), ((), ())),
                        preferred_element_type=jnp.float32)
    h = (b * jax.nn.sigmoid(b) * a).astype(jnp.bfloat16)
    y = lax.dot_general(h, w2_ref[0].astype(jnp.bfloat16),
                        (((1,), (0,)), ((), ())),
                        preferred_element_type=jnp.float32)
    yb = (y * col).astype(jnp.bfloat16)
    ylo = lax.bitcast_convert_type(yb[:, :D // 2], jnp.uint16).astype(jnp.uint32)
    yhi = lax.bitcast_convert_type(yb[:, D // 2:], jnp.uint16).astype(jnp.uint32)
    yw_ref[...] = lax.bitcast_convert_type(ylo | (yhi << 16), jnp.int32)


def _ffn(xg, wrt, Wl1, Wl2):
    return pl.pallas_call(
        _ffn_body,
        grid=(E,),
        in_specs=[
            pl.BlockSpec((C, D // 2), lambda e: (e, 0)),
            pl.BlockSpec((C, E + 1), lambda e: (0, 0)),
            pl.BlockSpec((1, D, 2 * H), lambda e: (e, 0, 0)),
            pl.BlockSpec((1, H, D), lambda e: (e, 0, 0)),
        ],
        out_specs=pl.BlockSpec((C, D // 2), lambda e: (e, 0)),
        out_shape=jax.ShapeDtypeStruct((E * C, D // 2), jnp.int32),
    )(xg, wrt, Wl1, Wl2)


def _add_chunk(buf0, buf1, obuf):
    himask = jnp.int32(-65536)

    @plsc.parallel_loop(0, CH, 1, unroll=2)
    def _(i):
        for j in range(D // 2 // 16):
            sl = pl.ds(j * 16, 16)
            v0 = buf0[i, sl]
            v1 = buf1[i, sl]
            lo = (lax.bitcast_convert_type(v0 << 16, jnp.float32) +
                  lax.bitcast_convert_type(v1 << 16, jnp.float32))
            hi = (lax.bitcast_convert_type(v0 & himask, jnp.float32) +
                  lax.bitcast_convert_type(v1 & himask, jnp.float32))
            obuf[i, sl] = lo
            obuf[i, pl.ds(D // 2 + j * 16, 16)] = hi


def _combine_body(yw_hbm, ridx_hbm, out_hbm, idx0a, idx1a, idx0b, idx1b,
                  b0a, b1a, b0b, b1b, oba, obb, sema, semb, semo):
    wid = lax.axis_index("s") * 2 + lax.axis_index("c")
    base0 = wid * TPW
    base1 = base0 + CH
    gathers = []
    for base, i0, i1, b0, b1, sem in ((base0, idx0a, idx1a, b0a, b1a, sema),
                                      (base1, idx0b, idx1b, b0b, b1b, semb)):
        pltpu.sync_copy(ridx_hbm.at[pl.ds(base, CH)], i0)
        pltpu.sync_copy(ridx_hbm.at[pl.ds(M + base, CH)], i1)
        cp0 = pltpu.make_async_copy(yw_hbm.at[i0], b0, sem)
        cp1 = pltpu.make_async_copy(yw_hbm.at[i1], b1, sem)
        cp0.start()
        cp1.start()
        gathers += [cp0, cp1]
    gathers[0].wait()
    gathers[1].wait()
    _add_chunk(b0a, b1a, oba)
    co0 = pltpu.make_async_copy(oba, out_hbm.at[pl.ds(base0, CH)], semo)
    co0.start()
    gathers[2].wait()
    gathers[3].wait()
    _add_chunk(b0b, b1b, obb)
    co1 = pltpu.make_async_copy(obb, out_hbm.at[pl.ds(base1, CH)], semo)
    co1.start()
    co0.wait()
    co1.wait()


@functools.cache
def _combine():
    return pl.kernel(
        _combine_body,
        out_type=jax.ShapeDtypeStruct((M, D), jnp.float32),
        mesh=plsc.VectorSubcoreMesh(core_axis_name="c", subcore_axis_name="s"),
        scratch_types=[
            pltpu.VMEM((CH,), jnp.int32),
            pltpu.VMEM((CH,), jnp.int32),
            pltpu.VMEM((CH,), jnp.int32),
            pltpu.VMEM((CH,), jnp.int32),
            pltpu.VMEM((CH, D // 2), jnp.int32),
            pltpu.VMEM((CH, D // 2), jnp.int32),
            pltpu.VMEM((CH, D // 2), jnp.int32),
            pltpu.VMEM((CH, D // 2), jnp.int32),
            pltpu.VMEM((CH, D), jnp.float32),
            pltpu.VMEM((CH, D), jnp.float32),
            pltpu.SemaphoreType.DMA,
            pltpu.SemaphoreType.DMA,
            pltpu.SemaphoreType.DMA,
        ],
    )


def kernel(x, Wg, Wl1, Wl2):
    ridx, wrt, xbf = _router(x, Wg)
    ridx_flat = ridx.reshape(2 * M)
    xg = _dispatch()(xbf, ridx_flat)
    yw = _ffn(xg, wrt, Wl1, Wl2)
    return _combine()(yw, ridx_flat)


# speedup vs baseline: 5.5586x; 1.0055x over previous
import functools

import jax
import jax.numpy as jnp
from jax import lax
from jax.experimental import pallas as pl
from jax.experimental.pallas import tpu as pltpu
from jax.experimental.pallas import tpu_sc as plsc

D = 768
H = 1024
E = 64
M = 2048
C = 128
HC = 1024
NHC = H // HC
NW = 32
TPW = M // NW
CH = 32


def _router_body(x_ref, wg_ref, ridx_ref, wrt_ref, xp_ref):
    x = x_ref[...]
    xb = x.astype(jnp.bfloat16)
    lo = lax.bitcast_convert_type(xb[:, :D // 2], jnp.uint16).astype(jnp.uint32)
    hi = lax.bitcast_convert_type(xb[:, D // 2:], jnp.uint16).astype(jnp.uint32)
    xp_ref[...] = lax.bitcast_convert_type(lo | (hi << 16), jnp.int32)
    logits = lax.dot_general(x, wg_ref[...], (((1,), (0,)), ((), ())),
                             preferred_element_type=jnp.float32)
    it = lax.broadcasted_iota(jnp.int32, (M, E), 1)
    m1 = jnp.max(logits, axis=1, keepdims=True)
    i1 = jnp.min(jnp.where(logits == m1, it, E), axis=1, keepdims=True)
    l2 = jnp.where(it == i1, -jnp.inf, logits)
    m2 = jnp.max(l2, axis=1, keepdims=True)
    i2 = jnp.min(jnp.where(l2 == m2, it, E), axis=1, keepdims=True)
    w2 = jax.nn.sigmoid(m2 - m1)
    w1 = 1.0 - w2

    one = jnp.float32(1.0)
    zero = jnp.float32(0.0)
    cnt = jnp.where(it == i1, one, zero) + jnp.where(it == i2, one, zero)

    B = 256
    rio = lax.broadcasted_iota(jnp.int32, (B, B), 0)
    cio = lax.broadcasted_iota(jnp.int32, (B, B), 1)
    tri = jnp.where(rio > cio, one, zero)
    carry = jnp.zeros((1, E), jnp.float32)
    blocks = []
    for b in range(M // B):
        blk = lax.slice(cnt, (b * B, 0), ((b + 1) * B, E))
        blocks.append(
            lax.dot_general(tri, blk, (((1,), (0,)), ((), ())),
                            preferred_element_type=jnp.float32) + carry)
        carry = carry + jnp.sum(blk, axis=0, keepdims=True)
    csum = jnp.concatenate(blocks, axis=0)

    s1 = jnp.sum(jnp.where(it == i1, csum, zero), axis=1).astype(jnp.int32)
    s2 = jnp.sum(jnp.where(it == i2, csum, zero), axis=1).astype(jnp.int32)
    e1 = i1[:, 0]
    e2 = i2[:, 0]
    cmin = jnp.min(carry, axis=1, keepdims=True)
    eit = lax.broadcasted_iota(jnp.int32, (1, E), 1)
    emin = jnp.min(jnp.where(carry == cmin, eit, E), axis=1)
    free_row = emin[0] * C + cmin[0, 0].astype(jnp.int32)
    r1 = jnp.where(s1 < C, e1 * C + s1, free_row)
    r2 = jnp.where(s2 < C, e2 * C + s2, free_row)
    ridx_ref[...] = jnp.concatenate(
        [r1.reshape(1, M), r2.reshape(1, M)], axis=0)

    itE = lax.broadcasted_iota(jnp.int32, (M, E + 1), 1)
    itC = lax.broadcasted_iota(jnp.int32, (M, C), 1)
    lhs1 = jnp.where(itE == i1, w1, zero)
    lhs2 = jnp.where(itE == i2, w2, zero)
    rhs1 = jnp.where(itC == s1[:, None], one, zero)
    rhs2 = jnp.where(itC == s2[:, None], one, zero)
    wrt = (lax.dot_general(rhs1, lhs1, (((0,), (0,)), ((), ())),
                           preferred_element_type=jnp.float32) +
           lax.dot_general(rhs2, lhs2, (((0,), (0,)), ((), ())),
                           preferred_element_type=jnp.float32))
    wrt_ref[...] = wrt


def _router(x, Wg):
    return pl.pallas_call(
        _router_body,
        out_shape=(jax.ShapeDtypeStruct((2, M), jnp.int32),
                   jax.ShapeDtypeStruct((C, E + 1), jnp.float32),
                   jax.ShapeDtypeStruct((M, D // 2), jnp.int32)),
    )(x, Wg)


def _dispatch_body(x_hbm, ridx_hbm, xg_hbm, idx0a, idx1a, idx0b, idx1b,
                   xba, xbb, sem):
    wid = lax.axis_index("s") * 2 + lax.axis_index("c")
    cps = []
    for c, (i0, i1, xb) in enumerate(((idx0a, idx1a, xba),
                                      (idx0b, idx1b, xbb))):
        base = wid * TPW + c * CH
        pltpu.sync_copy(ridx_hbm.at[pl.ds(base, CH)], i0)
        pltpu.sync_copy(ridx_hbm.at[pl.ds(M + base, CH)], i1)
        pltpu.sync_copy(x_hbm.at[pl.ds(base, CH)], xb)
        cp0 = pltpu.make_async_copy(xb, xg_hbm.at[i0], sem)
        cp1 = pltpu.make_async_copy(xb, xg_hbm.at[i1], sem)
        cp0.start()
        cp1.start()
        cps += [cp0, cp1]
    for cp in cps:
        cp.wait()


@functools.cache
def _dispatch():
    return pl.kernel(
        _dispatch_body,
        out_type=jax.ShapeDtypeStruct((E * C, D // 2), jnp.int32),
        mesh=plsc.VectorSubcoreMesh(core_axis_name="c", subcore_axis_name="s"),
        scratch_types=[
            pltpu.VMEM((CH,), jnp.int32),
            pltpu.VMEM((CH,), jnp.int32),
            pltpu.VMEM((CH,), jnp.int32),
            pltpu.VMEM((CH,), jnp.int32),
            pltpu.VMEM((CH, D // 2), jnp.int32),
            pltpu.VMEM((CH, D // 2), jnp.int32),
            pltpu.SemaphoreType.DMA,
        ],
    )


def _ffn_body(xg_ref, wrt_ref, w1_ref, w2_ref, yw_ref):
    e = pl.program_id(0)
    lane = lax.broadcasted_iota(jnp.int32, (C, E + 1), 1)
    col = jnp.sum(jnp.where(lane == e, wrt_ref[...], 0.0), axis=1,
                  keepdims=True)
    xp = lax.bitcast_convert_type(xg_ref[...], jnp.uint32)
    lo = lax.bitcast_convert_type(
        (xp & 0xFFFF).astype(jnp.uint16), jnp.bfloat16)
    hi = lax.bitcast_convert_type(
        (xp >> 16).astype(jnp.uint16), jnp.bfloat16)
    xm = jnp.where(col > 0.0, jnp.concatenate([lo, hi], axis=1),
                   jnp.bfloat16(0.0))
    w1 = w1_ref[0].astype(jnp.bfloat16)
    a = lax.dot_general(xm, w1[:, :H], (((1,), (0,)), ((), ())),
                        preferred_element_type=jnp.float32)
    b = lax.dot_general(xm, w1[:, H:], (((1,), (0,)), ((), ())),
                        preferred_element_type=jnp.float32)
    h = (b * jax.nn.sigmoid(b) * a).astype(jnp.bfloat16)
    y = lax.dot_general(h, w2_ref[0].astype(jnp.bfloat16),
                        (((1,), (0,)), ((), ())),
                        preferred_element_type=jnp.float32)
    yb = (y * col).astype(jnp.bfloat16)
    ylo = lax.bitcast_convert_type(yb[:, :D // 2], jnp.uint16).astype(jnp.uint32)
    yhi = lax.bitcast_convert_type(yb[:, D // 2:], jnp.uint16).astype(jnp.uint32)
    yw_ref[...] = lax.bitcast_convert_type(ylo | (yhi << 16), jnp.int32)


def _ffn(xg, wrt, Wl1, Wl2):
    return pl.pallas_call(
        _ffn_body,
        grid=(E,),
        in_specs=[
            pl.BlockSpec((C, D // 2), lambda e: (e, 0)),
            pl.BlockSpec((C, E + 1), lambda e: (0, 0)),
            pl.BlockSpec((1, D, 2 * H), lambda e: (e, 0, 0)),
            pl.BlockSpec((1, H, D), lambda e: (e, 0, 0)),
        ],
        out_specs=pl.BlockSpec((C, D // 2), lambda e: (e, 0)),
        out_shape=jax.ShapeDtypeStruct((E * C, D // 2), jnp.int32),
    )(xg, wrt, Wl1, Wl2)


def _add_chunk(buf0, buf1, obuf):
    himask = jnp.int32(-65536)

    @plsc.parallel_loop(0, CH, 1, unroll=4)
    def _(i):
        for j in range(D // 2 // 16):
            sl = pl.ds(j * 16, 16)
            v0 = buf0[i, sl]
            v1 = buf1[i, sl]
            lo = (lax.bitcast_convert_type(v0 << 16, jnp.float32) +
                  lax.bitcast_convert_type(v1 << 16, jnp.float32))
            hi = (lax.bitcast_convert_type(v0 & himask, jnp.float32) +
                  lax.bitcast_convert_type(v1 & himask, jnp.float32))
            obuf[i, sl] = lo
            obuf[i, pl.ds(D // 2 + j * 16, 16)] = hi


def _combine_body(yw_hbm, ridx_hbm, out_hbm, idx0a, idx1a, idx0b, idx1b,
                  b0a, b1a, b0b, b1b, oba, obb, sema, semb, semo):
    wid = lax.axis_index("s") * 2 + lax.axis_index("c")
    base0 = wid * TPW
    base1 = base0 + CH
    gathers = []
    for base, i0, i1, b0, b1, sem in ((base0, idx0a, idx1a, b0a, b1a, sema),
                                      (base1, idx0b, idx1b, b0b, b1b, semb)):
        pltpu.sync_copy(ridx_hbm.at[pl.ds(base, CH)], i0)
        pltpu.sync_copy(ridx_hbm.at[pl.ds(M + base, CH)], i1)
        cp0 = pltpu.make_async_copy(yw_hbm.at[i0], b0, sem)
        cp1 = pltpu.make_async_copy(yw_hbm.at[i1], b1, sem)
        cp0.start()
        cp1.start()
        gathers += [cp0, cp1]
    gathers[0].wait()
    gathers[1].wait()
    _add_chunk(b0a, b1a, oba)
    co0 = pltpu.make_async_copy(oba, out_hbm.at[pl.ds(base0, CH)], semo)
    co0.start()
    gathers[2].wait()
    gathers[3].wait()
    _add_chunk(b0b, b1b, obb)
    co1 = pltpu.make_async_copy(obb, out_hbm.at[pl.ds(base1, CH)], semo)
    co1.start()
    co0.wait()
    co1.wait()


@functools.cache
def _combine():
    return pl.kernel(
        _combine_body,
        out_type=jax.ShapeDtypeStruct((M, D), jnp.float32),
        mesh=plsc.VectorSubcoreMesh(core_axis_name="c", subcore_axis_name="s"),
        scratch_types=[
            pltpu.VMEM((CH,), jnp.int32),
            pltpu.VMEM((CH,), jnp.int32),
            pltpu.VMEM((CH,), jnp.int32),
            pltpu.VMEM((CH,), jnp.int32),
            pltpu.VMEM((CH, D // 2), jnp.int32),
            pltpu.VMEM((CH, D // 2), jnp.int32),
            pltpu.VMEM((CH, D // 2), jnp.int32),
            pltpu.VMEM((CH, D // 2), jnp.int32),
            pltpu.VMEM((CH, D), jnp.float32),
            pltpu.VMEM((CH, D), jnp.float32),
            pltpu.SemaphoreType.DMA,
            pltpu.SemaphoreType.DMA,
            pltpu.SemaphoreType.DMA,
        ],
    )


def kernel(x, Wg, Wl1, Wl2):
    ridx, wrt, xbf = _router(x, Wg)
    ridx_flat = ridx.reshape(2 * M)
    xg = _dispatch()(xbf, ridx_flat)
    yw = _ffn(xg, wrt, Wl1, Wl2)
    return _combine()(yw, ridx_flat)


# speedup vs baseline: 5.5953x; 1.0066x over previous
import functools

import jax
import jax.numpy as jnp
from jax import lax
from jax.experimental import pallas as pl
from jax.experimental.pallas import tpu as pltpu
from jax.experimental.pallas import tpu_sc as plsc

D = 768
H = 1024
E = 64
M = 2048
C = 128
HC = 1024
NHC = H // HC
NW = 32
TPW = M // NW
CH = 32


def _router_body(x_ref, wg_ref, ridx_ref, wrt_ref, xp_ref):
    x = x_ref[...]
    xb = x.astype(jnp.bfloat16)
    lo = lax.bitcast_convert_type(xb[:, :D // 2], jnp.uint16).astype(jnp.uint32)
    hi = lax.bitcast_convert_type(xb[:, D // 2:], jnp.uint16).astype(jnp.uint32)
    xp_ref[...] = lax.bitcast_convert_type(lo | (hi << 16), jnp.int32)
    logits = lax.dot_general(x, wg_ref[...], (((1,), (0,)), ((), ())),
                             preferred_element_type=jnp.float32)
    it = lax.broadcasted_iota(jnp.int32, (M, E), 1)
    m1 = jnp.max(logits, axis=1, keepdims=True)
    i1 = jnp.min(jnp.where(logits == m1, it, E), axis=1, keepdims=True)
    l2 = jnp.where(it == i1, -jnp.inf, logits)
    m2 = jnp.max(l2, axis=1, keepdims=True)
    i2 = jnp.min(jnp.where(l2 == m2, it, E), axis=1, keepdims=True)
    w2 = jax.nn.sigmoid(m2 - m1)
    w1 = 1.0 - w2

    one = jnp.float32(1.0)
    zero = jnp.float32(0.0)
    cnt = jnp.where(it == i1, one, zero) + jnp.where(it == i2, one, zero)

    B = 256
    rio = lax.broadcasted_iota(jnp.int32, (B, B), 0)
    cio = lax.broadcasted_iota(jnp.int32, (B, B), 1)
    tri = jnp.where(rio > cio, one, zero)
    carry = jnp.zeros((1, E), jnp.float32)
    blocks = []
    for b in range(M // B):
        blk = lax.slice(cnt, (b * B, 0), ((b + 1) * B, E))
        blocks.append(
            lax.dot_general(tri, blk, (((1,), (0,)), ((), ())),
                            preferred_element_type=jnp.float32) + carry)
        carry = carry + jnp.sum(blk, axis=0, keepdims=True)
    csum = jnp.concatenate(blocks, axis=0)

    s1 = jnp.sum(jnp.where(it == i1, csum, zero), axis=1).astype(jnp.int32)
    s2 = jnp.sum(jnp.where(it == i2, csum, zero), axis=1).astype(jnp.int32)
    e1 = i1[:, 0]
    e2 = i2[:, 0]
    cmin = jnp.min(carry, axis=1, keepdims=True)
    eit = lax.broadcasted_iota(jnp.int32, (1, E), 1)
    emin = jnp.min(jnp.where(carry == cmin, eit, E), axis=1)
    free_row = emin[0] * C + cmin[0, 0].astype(jnp.int32)
    r1 = jnp.where(s1 < C, e1 * C + s1, free_row)
    r2 = jnp.where(s2 < C, e2 * C + s2, free_row)
    ridx_ref[...] = jnp.concatenate(
        [r1.reshape(1, M), r2.reshape(1, M)], axis=0)

    itE = lax.broadcasted_iota(jnp.int32, (M, E + 1), 1)
    itC = lax.broadcasted_iota(jnp.int32, (M, C), 1)
    lhs1 = jnp.where(itE == i1, w1, zero)
    lhs2 = jnp.where(itE == i2, w2, zero)
    rhs1 = jnp.where(itC == s1[:, None], one, zero)
    rhs2 = jnp.where(itC == s2[:, None], one, zero)
    wrt = (lax.dot_general(rhs1, lhs1, (((0,), (0,)), ((), ())),
                           preferred_element_type=jnp.float32) +
           lax.dot_general(rhs2, lhs2, (((0,), (0,)), ((), ())),
                           preferred_element_type=jnp.float32))
    wrt_ref[...] = wrt


def _router(x, Wg):
    return pl.pallas_call(
        _router_body,
        out_shape=(jax.ShapeDtypeStruct((2, M), jnp.int32),
                   jax.ShapeDtypeStruct((C, E + 1), jnp.float32),
                   jax.ShapeDtypeStruct((M, D // 2), jnp.int32)),
    )(x, Wg)


def _dispatch_body(x_hbm, ridx_hbm, xg_hbm, idx0a, idx1a, idx0b, idx1b,
                   xba, xbb, sem):
    wid = lax.axis_index("s") * 2 + lax.axis_index("c")
    chunks = ((idx0a, idx1a, xba), (idx0b, idx1b, xbb))
    stages = []
    for c, (i0, i1, xb) in enumerate(chunks):
        base = wid * TPW + c * CH
        for src, dst in ((ridx_hbm.at[pl.ds(base, CH)], i0),
                         (ridx_hbm.at[pl.ds(M + base, CH)], i1),
                         (x_hbm.at[pl.ds(base, CH)], xb)):
            cp = pltpu.make_async_copy(src, dst, sem)
            cp.start()
            stages.append(cp)
    for cp in stages:
        cp.wait()
    cps = []
    for i0, i1, xb in chunks:
        cp0 = pltpu.make_async_copy(xb, xg_hbm.at[i0], sem)
        cp1 = pltpu.make_async_copy(xb, xg_hbm.at[i1], sem)
        cp0.start()
        cp1.start()
        cps += [cp0, cp1]
    for cp in cps:
        cp.wait()


@functools.cache
def _dispatch():
    return pl.kernel(
        _dispatch_body,
        out_type=jax.ShapeDtypeStruct((E * C, D // 2), jnp.int32),
        mesh=plsc.VectorSubcoreMesh(core_axis_name="c", subcore_axis_name="s"),
        scratch_types=[
            pltpu.VMEM((CH,), jnp.int32),
            pltpu.VMEM((CH,), jnp.int32),
            pltpu.VMEM((CH,), jnp.int32),
            pltpu.VMEM((CH,), jnp.int32),
            pltpu.VMEM((CH, D // 2), jnp.int32),
            pltpu.VMEM((CH, D // 2), jnp.int32),
            pltpu.SemaphoreType.DMA,
        ],
    )


def _ffn_body(xg_ref, wrt_ref, w1_ref, w2_ref, yw_ref):
    e = pl.program_id(0)
    lane = lax.broadcasted_iota(jnp.int32, (C, E + 1), 1)
    col = jnp.sum(jnp.where(lane == e, wrt_ref[...], 0.0), axis=1,
                  keepdims=True)
    xp = lax.bitcast_convert_type(xg_ref[...], jnp.uint32)
    lo = lax.bitcast_convert_type(
        (xp & 0xFFFF).astype(jnp.uint16), jnp.bfloat16)
    hi = lax.bitcast_convert_type(
        (xp >> 16).astype(jnp.uint16), jnp.bfloat16)
    xm = jnp.where(col > 0.0, jnp.concatenate([lo, hi], axis=1),
                   jnp.bfloat16(0.0))
    w1 = w1_ref[0].astype(jnp.bfloat16)
    a = lax.dot_general(xm, w1[:, :H], (((1,), (0,)), ((), ())),
                        preferred_element_type=jnp.float32)
    b = lax.dot_general(xm, w1[:, H:], (((1,), (0,)), ((), ())),
                        preferred_element_type=jnp.float32)
    h = (b * jax.nn.sigmoid(b) * a).astype(jnp.bfloat16)
    y = lax.dot_general(h, w2_ref[0].astype(jnp.bfloat16),
                        (((1,), (0,)), ((), ())),
                        preferred_element_type=jnp.float32)
    yb = (y * col).astype(jnp.bfloat16)
    ylo = lax.bitcast_convert_type(yb[:, :D // 2], jnp.uint16).astype(jnp.uint32)
    yhi = lax.bitcast_convert_type(yb[:, D // 2:], jnp.uint16).astype(jnp.uint32)
    yw_ref[...] = lax.bitcast_convert_type(ylo | (yhi << 16), jnp.int32)


def _ffn(xg, wrt, Wl1, Wl2):
    return pl.pallas_call(
        _ffn_body,
        grid=(E,),
        in_specs=[
            pl.BlockSpec((C, D // 2), lambda e: (e, 0)),
            pl.BlockSpec((C, E + 1), lambda e: (0, 0)),
            pl.BlockSpec((1, D, 2 * H), lambda e: (e, 0, 0)),
            pl.BlockSpec((1, H, D), lambda e: (e, 0, 0)),
        ],
        out_specs=pl.BlockSpec((C, D // 2), lambda e: (e, 0)),
        out_shape=jax.ShapeDtypeStruct((E * C, D // 2), jnp.int32),
    )(xg, wrt, Wl1, Wl2)


def _add_chunk(buf0, buf1, obuf):
    himask = jnp.int32(-65536)

    @plsc.parallel_loop(0, CH, 1, unroll=4)
    def _(i):
        for j in range(D // 2 // 16):
            sl = pl.ds(j * 16, 16)
            v0 = buf0[i, sl]
            v1 = buf1[i, sl]
            lo = (lax.bitcast_convert_type(v0 << 16, jnp.float32) +
                  lax.bitcast_convert_type(v1 << 16, jnp.float32))
            hi = (lax.bitcast_convert_type(v0 & himask, jnp.float32) +
                  lax.bitcast_convert_type(v1 & himask, jnp.float32))
            obuf[i, sl] = lo
            obuf[i, pl.ds(D // 2 + j * 16, 16)] = hi


def _combine_body(yw_hbm, ridx_hbm, out_hbm, idx0a, idx1a, idx0b, idx1b,
                  b0a, b1a, b0b, b1b, oba, obb, sema, semb, semo):
    wid = lax.axis_index("s") * 2 + lax.axis_index("c")
    base0 = wid * TPW
    base1 = base0 + CH
    chunks = ((base0, idx0a, idx1a, b0a, b1a, sema),
              (base1, idx0b, idx1b, b0b, b1b, semb))
    stages = []
    for base, i0, i1, b0, b1, sem in chunks:
        for src, dst in ((ridx_hbm.at[pl.ds(base, CH)], i0),
                         (ridx_hbm.at[pl.ds(M + base, CH)], i1)):
            cp = pltpu.make_async_copy(src, dst, semo)
            cp.start()
            stages.append(cp)
    for cp in stages:
        cp.wait()
    gathers = []
    for base, i0, i1, b0, b1, sem in chunks:
        cp0 = pltpu.make_async_copy(yw_hbm.at[i0], b0, sem)
        cp1 = pltpu.make_async_copy(yw_hbm.at[i1], b1, sem)
        cp0.start()
        cp1.start()
        gathers += [cp0, cp1]
    gathers[0].wait()
    gathers[1].wait()
    _add_chunk(b0a, b1a, oba)
    co0 = pltpu.make_async_copy(oba, out_hbm.at[pl.ds(base0, CH)], semo)
    co0.start()
    gathers[2].wait()
    gathers[3].wait()
    _add_chunk(b0b, b1b, obb)
    co1 = pltpu.make_async_copy(obb, out_hbm.at[pl.ds(base1, CH)], semo)
    co1.start()
    co0.wait()
    co1.wait()


@functools.cache
def _combine():
    return pl.kernel(
        _combine_body,
        out_type=jax.ShapeDtypeStruct((M, D), jnp.float32),
        mesh=plsc.VectorSubcoreMesh(core_axis_name="c", subcore_axis_name="s"),
        scratch_types=[
            pltpu.VMEM((CH,), jnp.int32),
            pltpu.VMEM((CH,), jnp.int32),
            pltpu.VMEM((CH,), jnp.int32),
            pltpu.VMEM((CH,), jnp.int32),
            pltpu.VMEM((CH, D // 2), jnp.int32),
            pltpu.VMEM((CH, D // 2), jnp.int32),
            pltpu.VMEM((CH, D // 2), jnp.int32),
            pltpu.VMEM((CH, D // 2), jnp.int32),
            pltpu.VMEM((CH, D), jnp.float32),
            pltpu.VMEM((CH, D), jnp.float32),
            pltpu.SemaphoreType.DMA,
            pltpu.SemaphoreType.DMA,
            pltpu.SemaphoreType.DMA,
        ],
    )


def kernel(x, Wg, Wl1, Wl2):
    ridx, wrt, xbf = _router(x, Wg)
    ridx_flat = ridx.reshape(2 * M)
    xg = _dispatch()(xbf, ridx_flat)
    yw = _ffn(xg, wrt, Wl1, Wl2)
    return _combine()(yw, ridx_flat)
